# Initial kernel scaffold; baseline (speedup 1.0000x reference)
import jax
import jax.numpy as jnp
from jax.experimental import pallas as pl


def kernel(U, edge_index, A_values, W, B, V_w):
    raise NotImplementedError("write your pallas kernel here")


# speedup vs baseline: 1.4048x; 1.4048x over previous
import functools

import jax
import jax.numpy as jnp
from jax import lax
from jax.experimental import pallas as pl
from jax.experimental.pallas import tpu as pltpu
from jax.experimental.pallas import tpu_sc as plsc

N = 10000
E = 160000
NH = 128
NCLASS = 16
KAPPA = 0.99
THRESHOLD = 20
LAM = 1e-4

NP_ = 10240
TILES = 32
RPT = NP_ // TILES
CHUNK = 256
EA = E + TILES * CHUNK
BLK = 512

@functools.cache
def _mesh():
    return plsc.VectorSubcoreMesh(
        core_axis_name="c", subcore_axis_name="s", num_cores=2, num_subcores=16
    )


def _spmm_body(y_hbm, cols_hbm, vals_hbm, lrows_hbm, meta_hbm, zeros_hbm,
               out_hbm, cbuf, vbuf, rbuf, ybuf, acc, mbuf, sem):
    c = lax.axis_index("c")
    s = lax.axis_index("s")
    wid = s * 2 + c

    pltpu.sync_copy(meta_hbm.at[wid], mbuf)
    mv = mbuf[...]
    ch_lo = mv[0]
    ch_hi = mv[1]

    pltpu.sync_copy(zeros_hbm, acc)

    iota = lax.iota(jnp.int32, 16)

    def chunk_body(ch, _):
        pltpu.sync_copy(cols_hbm.at[pl.ds(ch * 2, 2)], cbuf)
        pltpu.sync_copy(vals_hbm.at[pl.ds(ch * CHUNK, CHUNK)], vbuf)
        pltpu.sync_copy(lrows_hbm.at[pl.ds(ch * CHUNK, CHUNK)], rbuf)
        cp0 = pltpu.async_copy(y_hbm.at[cbuf.at[0]], ybuf.at[pl.ds(0, 128)], sem)
        cp1 = pltpu.async_copy(y_hbm.at[cbuf.at[1]], ybuf.at[pl.ds(128, 128)], sem)
        cp0.wait()
        cp1.wait()

        def group_body(g, _):
            vv = vbuf[pl.ds(g * 16, 16)]
            lr = rbuf[pl.ds(g * 16, 16)]
            for l in range(16):
                e = g * 16 + l
                r = lr[l]
                ve = vv[l]
                for j in range(8):
                    x = ybuf[e, pl.ds(j * 16, 16)]
                    plsc.addupdate(acc.at[r, pl.ds(j * 16, 16)], x * ve)
            return 0

        lax.fori_loop(0, CHUNK // 16, group_body, 0)
        return 0

    lax.fori_loop(ch_lo, ch_hi, chunk_body, 0)
    pltpu.sync_copy(acc, out_hbm.at[pl.ds(wid * RPT, RPT)])


@functools.cache
def _spmm_kernel():
    return pl.kernel(
        _spmm_body,
        out_type=jax.ShapeDtypeStruct((NP_, NH), jnp.float32),
        mesh=_mesh(),
        scratch_types=[
            pltpu.VMEM((2, 128), jnp.int32),
            pltpu.VMEM((CHUNK,), jnp.float32),
            pltpu.VMEM((CHUNK,), jnp.int32),
            pltpu.VMEM((CHUNK, NH), jnp.float32),
            pltpu.VMEM((RPT, NH), jnp.float32),
            pltpu.VMEM((16,), jnp.int32),
            pltpu.SemaphoreType.DMA,
        ],
    )


def _spmm_call(*args):
    return _spmm_kernel()(*args)


def _prop_body(f_ref, a_ref, w_ref, xk_ref, y_ref):
    F = f_ref[...]
    al = a_ref[...]
    xk = jnp.sum(F * al[:5][:, None, :], axis=0)
    xk_ref[...] = xk
    y_ref[...] = jnp.dot(xk, w_ref[...], preferred_element_type=jnp.float32)


def _prop_call(f_hist, a8, wp):
    return pl.pallas_call(
        _prop_body,
        grid=(NP_ // BLK,),
        in_specs=[
            pl.BlockSpec((5, BLK, NH), lambda i: (0, i, 0)),
            pl.BlockSpec((8, NH), lambda i: (0, 0)),
            pl.BlockSpec((NH, NH), lambda i: (0, 0)),
        ],
        out_specs=[
            pl.BlockSpec((BLK, NH), lambda i: (i, 0)),
            pl.BlockSpec((BLK, NH), lambda i: (i, 0)),
        ],
        out_shape=[
            jax.ShapeDtypeStruct((NP_, NH), jnp.float32),
            jax.ShapeDtypeStruct((NP_, NH), jnp.float32),
        ],
    )(f_hist, a8, wp)


def _comb_body(p_ref, aub_ref, xk_ref, g_ref, z_ref, gn_ref, d_ref):
    z = jnp.maximum(p_ref[...] + aub_ref[...], 0.0)
    gnew = z - xk_ref[...]
    z_ref[...] = z
    gn_ref[...] = gnew
    G = g_ref[...]
    parts = jnp.sum(G * gnew[None], axis=1)
    selfp = jnp.sum(gnew * gnew, axis=0)[None]
    d = jnp.concatenate([parts, selfp, jnp.zeros((2, NH), jnp.float32)], axis=0)

    @pl.when(pl.program_id(0) == 0)
    def _():
        d_ref[...] = jnp.zeros_like(d_ref)

    d_ref[...] += d


def _comb_call(p, aub, xk, g_hist):
    return pl.pallas_call(
        _comb_body,
        grid=(NP_ // BLK,),
        in_specs=[
            pl.BlockSpec((BLK, NH), lambda i: (i, 0)),
            pl.BlockSpec((BLK, NH), lambda i: (i, 0)),
            pl.BlockSpec((BLK, NH), lambda i: (i, 0)),
            pl.BlockSpec((5, BLK, NH), lambda i: (0, i, 0)),
        ],
        out_specs=[
            pl.BlockSpec((BLK, NH), lambda i: (i, 0)),
            pl.BlockSpec((BLK, NH), lambda i: (i, 0)),
            pl.BlockSpec((8, NH), lambda i: (0, 0)),
        ],
        out_shape=[
            jax.ShapeDtypeStruct((NP_, NH), jnp.float32),
            jax.ShapeDtypeStruct((NP_, NH), jnp.float32),
            jax.ShapeDtypeStruct((8, NH), jnp.float32),
        ],
    )(p, aub, xk, g_hist)


def _init_body(pu_ref, b_ref, aub_ref, f0_ref, d_ref):
    aub = jnp.dot(pu_ref[...], b_ref[...], preferred_element_type=jnp.float32)
    f0 = jnp.maximum(aub, 0.0)
    aub_ref[...] = aub
    f0_ref[...] = f0
    d = jnp.concatenate(
        [jnp.sum(f0 * f0, axis=0)[None], jnp.zeros((7, NH), jnp.float32)], axis=0
    )

    @pl.when(pl.program_id(0) == 0)
    def _():
        d_ref[...] = jnp.zeros_like(d_ref)

    d_ref[...] += d


def _init_call(pu, b):
    return pl.pallas_call(
        _init_body,
        grid=(NP_ // BLK,),
        in_specs=[
            pl.BlockSpec((BLK, NH), lambda i: (i, 0)),
            pl.BlockSpec((NH, NH), lambda i: (0, 0)),
        ],
        out_specs=[
            pl.BlockSpec((BLK, NH), lambda i: (i, 0)),
            pl.BlockSpec((BLK, NH), lambda i: (i, 0)),
            pl.BlockSpec((8, NH), lambda i: (0, 0)),
        ],
        out_shape=[
            jax.ShapeDtypeStruct((NP_, NH), jnp.float32),
            jax.ShapeDtypeStruct((NP_, NH), jnp.float32),
            jax.ShapeDtypeStruct((8, NH), jnp.float32),
        ],
    )(pu, b)


def _proj_body(w_ref, kap_ref, out_ref):
    W = w_ref[...]
    kapc = kap_ref[...][0:1, 0:1]
    a = jnp.abs(W)
    s = jnp.sum(a, axis=1, keepdims=True)
    hi0 = jnp.max(a, axis=1, keepdims=True)

    def bis(_, lh):
        lo, hi = lh
        mid = 0.5 * (lo + hi)
        t = jnp.sum(jnp.maximum(a - mid, 0.0), axis=1, keepdims=True)
        pred = t > kapc
        return jnp.where(pred, mid, lo), jnp.where(pred, hi, mid)

    lo, hi = lax.fori_loop(0, 60, bis, (jnp.zeros_like(s), hi0))
    theta = 0.5 * (lo + hi)
    proj = jnp.sign(W) * jnp.maximum(a - theta, 0.0)
    out_ref[...] = jnp.where(s > kapc, proj, W)


def _proj_call(w, kap):
    return pl.pallas_call(
        _proj_body,
        in_specs=[
            pl.BlockSpec((NH, NH), lambda: (0, 0)),
            pl.BlockSpec((8, NH), lambda: (0, 0)),
        ],
        out_specs=pl.BlockSpec((NH, NH), lambda: (0, 0)),
        out_shape=jax.ShapeDtypeStruct((NH, NH), jnp.float32),
    )(w, kap)


def _mm_body(x_ref, w_ref, o_ref):
    o_ref[...] = jnp.dot(x_ref[...], w_ref[...], preferred_element_type=jnp.float32)


def _mm_call(x, w):
    return pl.pallas_call(
        _mm_body,
        grid=(NP_ // BLK,),
        in_specs=[
            pl.BlockSpec((BLK, NH), lambda i: (i, 0)),
            pl.BlockSpec((NH, NH), lambda i: (0, 0)),
        ],
        out_specs=pl.BlockSpec((BLK, NH), lambda i: (i, 0)),
        out_shape=jax.ShapeDtypeStruct((NP_, NH), jnp.float32),
    )(x, w)


def _preprocess(edge_index, a_values):
    row = edge_index[0].astype(jnp.int32)
    col = edge_index[1].astype(jnp.int32)
    order = jnp.argsort(row)
    rs = row[order]
    cls = col[order]
    vs = a_values[order]

    t_e = rs // RPT
    cnt = jnp.bincount(t_e, length=TILES)
    nch = (cnt + CHUNK - 1) // CHUNK
    choff = jnp.concatenate([jnp.zeros((1,), jnp.int32),
                             jnp.cumsum(nch).astype(jnp.int32)])
    seg_start = choff * CHUNK
    cum_cnt = jnp.concatenate([jnp.zeros((1,), jnp.int32),
                               jnp.cumsum(cnt).astype(jnp.int32)])
    pos = seg_start[t_e] + (jnp.arange(E, dtype=jnp.int32) - cum_cnt[t_e])

    cols_p = jnp.zeros((EA,), jnp.int32).at[pos].set(cls)
    vals_p = jnp.zeros((EA,), jnp.float32).at[pos].set(vs)
    lrow_p = jnp.zeros((EA,), jnp.int32).at[pos].set(rs - t_e * RPT)

    meta = jnp.zeros((TILES, 16), jnp.int32)
    meta = meta.at[:, 0].set(choff[:-1])
    meta = meta.at[:, 1].set(choff[1:])

    cols2 = cols_p.reshape(EA // 128, 128)
    return cols2, vals_p, lrow_p, meta


def _spmm(y, pre):
    cols2, vals_p, lrow_p, meta = pre
    zeros = jnp.zeros((RPT, NH), jnp.float32)
    return _spmm_call(y, cols2, vals_p, lrow_p, meta, zeros)


def _spectral_rad(edge_index, values, iters=50):
    av = jnp.abs(values)
    row = edge_index[0]
    col = edge_index[1]
    v = jnp.ones((N,), jnp.float32) / jnp.sqrt(N)
    for _ in range(iters):
        w = jax.ops.segment_sum(av * v[col], row, num_segments=N)
        v = w / (jnp.linalg.norm(w) + 1e-12)
    w = jax.ops.segment_sum(av * v[col], row, num_segments=N)
    return jnp.linalg.norm(w) + 1e-5


def kernel(U, edge_index, A_values, W, B, V_w):
    pre = _preprocess(edge_index, A_values)

    rho = _spectral_rad(edge_index, A_values)
    kap_eff = KAPPA / rho
    kap_arr = jnp.full((8, NH), kap_eff, jnp.float32)
    Wp = _proj_call(W, kap_arr)

    u_pad = jnp.concatenate([U, jnp.zeros((NP_ - N, NH), jnp.float32)], axis=0)
    p_u = _spmm(u_pad, pre)
    aub, f0, d0 = _init_call(p_u, B)

    f_hist = jnp.zeros((5, NP_, NH), jnp.float32).at[0].set(f0)
    g_hist = jnp.zeros((5, NP_, NH), jnp.float32).at[0].set(f0)
    M = jnp.zeros((5, 5), jnp.float32).at[0, 0].set(jnp.sum(d0[0]))

    def step(alpha5, sl):
        a8 = jnp.zeros((8, NH), jnp.float32).at[:5, :].set(alpha5[:, None])
        xk, y = _prop_call(f_hist, a8, Wp)
        p = _spmm(y, pre)
        z, gn, d = _comb_call(p, aub, xk, g_hist)
        d6 = jnp.sum(d, axis=1)[:6]
        newrow = d6[:5].at[sl].set(d6[5])
        return z, gn, newrow

    alpha5 = jnp.zeros((5,), jnp.float32).at[0].set(1.0)
    z, gn, newrow = step(alpha5, 1)
    f_hist = f_hist.at[1].set(z)
    g_hist = g_hist.at[1].set(gn)
    M = M.at[1, :].set(newrow).at[:, 1].set(newrow)

    for k in range(2, THRESHOLD):
        nn = min(k, 5)
        sl = k % 5
        H = jnp.zeros((nn + 1, nn + 1), jnp.float32)
        H = H.at[0, 1:].set(1.0).at[1:, 0].set(1.0)
        H = H.at[1:, 1:].set(M[:nn, :nn] + LAM * jnp.eye(nn, dtype=jnp.float32))
        yv = jnp.zeros((nn + 1,), jnp.float32).at[0].set(1.0)
        alpha = jnp.linalg.solve(H, yv)[1:]
        alpha5 = jnp.zeros((5,), jnp.float32).at[:nn].set(alpha)
        z, gn, newrow = step(alpha5, sl)
        f_hist = f_hist.at[sl].set(z)
        g_hist = g_hist.at[sl].set(gn)
        M = M.at[sl, :].set(newrow).at[:, sl].set(newrow)

    z_star = f_hist[4]
    vwt = jnp.zeros((NH, NH), jnp.float32).at[:, :NCLASS].set(V_w.T)
    labels = _mm_call(z_star, vwt)
    return labels[:N, :NCLASS], z_star[:N]


# speedup vs baseline: 4.5372x; 3.2298x over previous
import functools

import jax
import jax.numpy as jnp
from jax import lax
from jax.experimental import pallas as pl
from jax.experimental.pallas import tpu as pltpu
from jax.experimental.pallas import tpu_sc as plsc

N = 10000
E = 160000
NH = 128
NCLASS = 16
KAPPA = 0.99
THRESHOLD = 20
LAM = 1e-4

NP_ = 10240
TILES = 32
RPT = NP_ // TILES
CHUNK = 256
EA = E + TILES * CHUNK
BLK = 512

@functools.cache
def _mesh():
    return plsc.VectorSubcoreMesh(
        core_axis_name="c", subcore_axis_name="s", num_cores=2, num_subcores=16
    )


def _spmm_body(y_hbm, cols_hbm, vals_hbm, lrows_hbm, meta_hbm, zeros_hbm,
               out_hbm, cbuf, vbuf, rbuf, ybuf, acc, mbuf, sem):
    c = lax.axis_index("c")
    s = lax.axis_index("s")
    wid = s * 2 + c

    pltpu.sync_copy(meta_hbm.at[wid], mbuf)
    mv = mbuf[...]
    ch_lo = mv[0]
    ch_hi = mv[1]

    pltpu.sync_copy(zeros_hbm, acc)

    iota = lax.iota(jnp.int32, 16)

    def chunk_body(ch, _):
        pltpu.sync_copy(cols_hbm.at[pl.ds(ch * 2, 2)], cbuf)
        pltpu.sync_copy(vals_hbm.at[pl.ds(ch * CHUNK, CHUNK)], vbuf)
        pltpu.sync_copy(lrows_hbm.at[pl.ds(ch * CHUNK, CHUNK)], rbuf)
        cp0 = pltpu.async_copy(y_hbm.at[cbuf.at[0]], ybuf.at[pl.ds(0, 128)], sem)
        cp1 = pltpu.async_copy(y_hbm.at[cbuf.at[1]], ybuf.at[pl.ds(128, 128)], sem)
        cp0.wait()
        cp1.wait()

        def group_body(g, _):
            vv = vbuf[pl.ds(g * 16, 16)]
            lr = rbuf[pl.ds(g * 16, 16)]
            for l in range(16):
                e = g * 16 + l
                r = lr[l]
                ve = vv[l]
                for j in range(8):
                    x = ybuf[e, pl.ds(j * 16, 16)]
                    plsc.addupdate(acc.at[r, pl.ds(j * 16, 16)], x * ve)
            return 0

        lax.fori_loop(0, CHUNK // 16, group_body, 0)
        return 0

    lax.fori_loop(ch_lo, ch_hi, chunk_body, 0)
    pltpu.sync_copy(acc, out_hbm.at[pl.ds(wid * RPT, RPT)])


@functools.cache
def _spmm_kernel():
    return pl.kernel(
        _spmm_body,
        out_type=jax.ShapeDtypeStruct((NP_, NH), jnp.float32),
        mesh=_mesh(),
        scratch_types=[
            pltpu.VMEM((2, 128), jnp.int32),
            pltpu.VMEM((CHUNK,), jnp.float32),
            pltpu.VMEM((CHUNK,), jnp.int32),
            pltpu.VMEM((CHUNK, NH), jnp.float32),
            pltpu.VMEM((RPT, NH), jnp.float32),
            pltpu.VMEM((16,), jnp.int32),
            pltpu.SemaphoreType.DMA,
        ],
    )


def _spmm_call(*args):
    return _spmm_kernel()(*args)


PROWS = NP_ // 16
_POWER_ARG_SHAPES = [
    ((EA,), jnp.int32),
    ((EA,), jnp.float32),
    ((EA,), jnp.int32),
    ((EA,), jnp.int32),
    ((EA,), jnp.int32),
    ((EA,), jnp.int32),
    ((16, 16), jnp.int32),
    ((NP_,), jnp.float32),
]


def _power_body(cols_hbm, avals_hbm, lr_hbm, rn_hbm, m1_hbm, m2_hbm,
                pmeta_hbm, v0_hbm, wout_hbm, vout_hbm,
                cb, vb, lrb, rnb, m1b, m2b, vref, wref, vseg, maxb, mb2,
                mbuf, sh_s, sh_v, sem):
    c = lax.axis_index("c")
    sid = lax.axis_index("s")

    @pl.when(c == 0)
    def _():
        pltpu.sync_copy(pmeta_hbm.at[sid], mbuf)
        mv = mbuf[...]
        ch_lo = mv[0]
        ch_hi = mv[1]
        pltpu.sync_copy(v0_hbm, vref)

        def matvec():
            def zero_body(i, _):
                wref[pl.ds(i * 16, 16)] = jnp.zeros((16,), jnp.float32)
                return 0
            lax.fori_loop(0, (PROWS + 16) // 16, zero_body, 0)

            def chunk_body(ch, _):
                pltpu.sync_copy(cols_hbm.at[pl.ds(ch * CHUNK, CHUNK)], cb)
                pltpu.sync_copy(avals_hbm.at[pl.ds(ch * CHUNK, CHUNK)], vb)
                pltpu.sync_copy(lr_hbm.at[pl.ds(ch * CHUNK, CHUNK)], lrb)
                pltpu.sync_copy(rn_hbm.at[pl.ds(ch * CHUNK, CHUNK)], rnb)
                pltpu.sync_copy(m1_hbm.at[pl.ds(ch * CHUNK, CHUNK)], m1b)
                pltpu.sync_copy(m2_hbm.at[pl.ds(ch * CHUNK, CHUNK)], m2b)

                def group_body(g, _):
                    sl = pl.ds(g * 16, 16)
                    x = plsc.load_gather(vref, [cb[sl]])
                    cs = plsc.cumsum(vb[sl] * x)
                    plsc.addupdate_scatter(wref, [lrb[sl]], cs,
                                           mask=m1b[sl] > 0)
                    plsc.addupdate_scatter(wref, [rnb[sl]], -cs,
                                           mask=m2b[sl] > 0)
                    return 0

                lax.fori_loop(0, CHUNK // 16, group_body, 0)
                return 0

            lax.fori_loop(ch_lo, ch_hi, chunk_body, 0)

        def iter_body(_, __):
            matvec()
            def max_body(i, m):
                return jnp.maximum(m, jnp.abs(wref[pl.ds(i * 16, 16)]))
            m = lax.fori_loop(0, PROWS // 16, max_body,
                              jnp.zeros((16,), jnp.float32))
            maxb[...] = jnp.full((16,), jnp.max(m), jnp.float32)
            pltpu.sync_copy(maxb, sh_s.at[sid])
            plsc.subcore_barrier()
            pltpu.sync_copy(sh_s, mb2)

            def gmax_body(t, m):
                return jnp.maximum(m, mb2[t])
            gm = lax.fori_loop(0, 16, gmax_body, jnp.zeros((16,), jnp.float32))
            sv = jnp.full((16,), jnp.max(gm), jnp.float32)
            invv = jnp.full((16,), 1.0, jnp.float32) / jnp.maximum(sv, 1e-30)

            def scale_body(i, _):
                vseg[pl.ds(i * 16, 16)] = wref[pl.ds(i * 16, 16)] * invv
                return 0
            lax.fori_loop(0, PROWS // 16, scale_body, 0)
            pltpu.sync_copy(vseg, sh_v.at[pl.ds(sid * PROWS, PROWS)])
            plsc.subcore_barrier()
            pltpu.sync_copy(sh_v, vref)
            plsc.subcore_barrier()
            return 0

        lax.fori_loop(0, 50, iter_body, 0)
        matvec()

        def out_body(i, _):
            vseg[pl.ds(i * 16, 16)] = wref[pl.ds(i * 16, 16)]
            return 0
        lax.fori_loop(0, PROWS // 16, out_body, 0)
        pltpu.sync_copy(vseg, wout_hbm.at[pl.ds(sid * PROWS, PROWS)])
        pltpu.sync_copy(vref.at[pl.ds(sid * PROWS, PROWS)],
                        vout_hbm.at[pl.ds(sid * PROWS, PROWS)])


@functools.cache
def _power_kernel():
    return pl.kernel(
        _power_body,
        out_type=(
            jax.ShapeDtypeStruct((NP_,), jnp.float32),
            jax.ShapeDtypeStruct((NP_,), jnp.float32),
        ),
        mesh=_mesh(),
        scratch_types=[
            pltpu.VMEM((CHUNK,), jnp.int32),
            pltpu.VMEM((CHUNK,), jnp.float32),
            pltpu.VMEM((CHUNK,), jnp.int32),
            pltpu.VMEM((CHUNK,), jnp.int32),
            pltpu.VMEM((CHUNK,), jnp.int32),
            pltpu.VMEM((CHUNK,), jnp.int32),
            pltpu.VMEM((NP_,), jnp.float32),
            pltpu.VMEM((PROWS + 16,), jnp.float32),
            pltpu.VMEM((PROWS,), jnp.float32),
            pltpu.VMEM((16,), jnp.float32),
            pltpu.VMEM((16, 16), jnp.float32),
            pltpu.VMEM((16,), jnp.int32),
            pltpu.VMEM_SHARED((16, 16), jnp.float32),
            pltpu.VMEM_SHARED((NP_,), jnp.float32),
            pltpu.SemaphoreType.DMA,
        ],
        compiler_params=pltpu.CompilerParams(needs_layout_passes=False),
    )


def _power_call(*args):
    return _power_kernel()(*args)


def _prop_body(f_ref, a_ref, w_ref, xk_ref, y_ref):
    F = f_ref[...]
    al = a_ref[...]
    xk = jnp.sum(F * al[:5][:, None, :], axis=0)
    xk_ref[...] = xk
    y_ref[...] = jnp.dot(xk, w_ref[...], preferred_element_type=jnp.float32)


def _prop_call(f_hist, a8, wp):
    return pl.pallas_call(
        _prop_body,
        grid=(NP_ // BLK,),
        in_specs=[
            pl.BlockSpec((5, BLK, NH), lambda i: (0, i, 0)),
            pl.BlockSpec((8, NH), lambda i: (0, 0)),
            pl.BlockSpec((NH, NH), lambda i: (0, 0)),
        ],
        out_specs=[
            pl.BlockSpec((BLK, NH), lambda i: (i, 0)),
            pl.BlockSpec((BLK, NH), lambda i: (i, 0)),
        ],
        out_shape=[
            jax.ShapeDtypeStruct((NP_, NH), jnp.float32),
            jax.ShapeDtypeStruct((NP_, NH), jnp.float32),
        ],
    )(f_hist, a8, wp)


def _comb_body(p_ref, aub_ref, xk_ref, g_ref, z_ref, gn_ref, d_ref):
    z = jnp.maximum(p_ref[...] + aub_ref[...], 0.0)
    gnew = z - xk_ref[...]
    z_ref[...] = z
    gn_ref[...] = gnew
    G = g_ref[...]
    parts = jnp.sum(G * gnew[None], axis=1)
    selfp = jnp.sum(gnew * gnew, axis=0)[None]
    d = jnp.concatenate([parts, selfp, jnp.zeros((2, NH), jnp.float32)], axis=0)

    @pl.when(pl.program_id(0) == 0)
    def _():
        d_ref[...] = jnp.zeros_like(d_ref)

    d_ref[...] += d


def _comb_call(p, aub, xk, g_hist):
    return pl.pallas_call(
        _comb_body,
        grid=(NP_ // BLK,),
        in_specs=[
            pl.BlockSpec((BLK, NH), lambda i: (i, 0)),
            pl.BlockSpec((BLK, NH), lambda i: (i, 0)),
            pl.BlockSpec((BLK, NH), lambda i: (i, 0)),
            pl.BlockSpec((5, BLK, NH), lambda i: (0, i, 0)),
        ],
        out_specs=[
            pl.BlockSpec((BLK, NH), lambda i: (i, 0)),
            pl.BlockSpec((BLK, NH), lambda i: (i, 0)),
            pl.BlockSpec((8, NH), lambda i: (0, 0)),
        ],
        out_shape=[
            jax.ShapeDtypeStruct((NP_, NH), jnp.float32),
            jax.ShapeDtypeStruct((NP_, NH), jnp.float32),
            jax.ShapeDtypeStruct((8, NH), jnp.float32),
        ],
    )(p, aub, xk, g_hist)


def _init_body(pu_ref, b_ref, aub_ref, f0_ref, d_ref):
    aub = jnp.dot(pu_ref[...], b_ref[...], preferred_element_type=jnp.float32)
    f0 = jnp.maximum(aub, 0.0)
    aub_ref[...] = aub
    f0_ref[...] = f0
    d = jnp.concatenate(
        [jnp.sum(f0 * f0, axis=0)[None], jnp.zeros((7, NH), jnp.float32)], axis=0
    )

    @pl.when(pl.program_id(0) == 0)
    def _():
        d_ref[...] = jnp.zeros_like(d_ref)

    d_ref[...] += d


def _init_call(pu, b):
    return pl.pallas_call(
        _init_body,
        grid=(NP_ // BLK,),
        in_specs=[
            pl.BlockSpec((BLK, NH), lambda i: (i, 0)),
            pl.BlockSpec((NH, NH), lambda i: (0, 0)),
        ],
        out_specs=[
            pl.BlockSpec((BLK, NH), lambda i: (i, 0)),
            pl.BlockSpec((BLK, NH), lambda i: (i, 0)),
            pl.BlockSpec((8, NH), lambda i: (0, 0)),
        ],
        out_shape=[
            jax.ShapeDtypeStruct((NP_, NH), jnp.float32),
            jax.ShapeDtypeStruct((NP_, NH), jnp.float32),
            jax.ShapeDtypeStruct((8, NH), jnp.float32),
        ],
    )(pu, b)


def _proj_body(w_ref, kap_ref, out_ref):
    W = w_ref[...]
    kapc = kap_ref[...][0:1, 0:1]
    a = jnp.abs(W)
    s = jnp.sum(a, axis=1, keepdims=True)
    hi0 = jnp.max(a, axis=1, keepdims=True)

    def bis(_, lh):
        lo, hi = lh
        mid = 0.5 * (lo + hi)
        t = jnp.sum(jnp.maximum(a - mid, 0.0), axis=1, keepdims=True)
        pred = t > kapc
        return jnp.where(pred, mid, lo), jnp.where(pred, hi, mid)

    lo, hi = lax.fori_loop(0, 60, bis, (jnp.zeros_like(s), hi0))
    theta = 0.5 * (lo + hi)
    proj = jnp.sign(W) * jnp.maximum(a - theta, 0.0)
    out_ref[...] = jnp.where(s > kapc, proj, W)


def _proj_call(w, kap):
    return pl.pallas_call(
        _proj_body,
        in_specs=[
            pl.BlockSpec((NH, NH), lambda: (0, 0)),
            pl.BlockSpec((8, NH), lambda: (0, 0)),
        ],
        out_specs=pl.BlockSpec((NH, NH), lambda: (0, 0)),
        out_shape=jax.ShapeDtypeStruct((NH, NH), jnp.float32),
    )(w, kap)


def _mm_body(x_ref, w_ref, o_ref):
    o_ref[...] = jnp.dot(x_ref[...], w_ref[...], preferred_element_type=jnp.float32)


def _mm_call(x, w):
    return pl.pallas_call(
        _mm_body,
        grid=(NP_ // BLK,),
        in_specs=[
            pl.BlockSpec((BLK, NH), lambda i: (i, 0)),
            pl.BlockSpec((NH, NH), lambda i: (0, 0)),
        ],
        out_specs=pl.BlockSpec((BLK, NH), lambda i: (i, 0)),
        out_shape=jax.ShapeDtypeStruct((NP_, NH), jnp.float32),
    )(x, w)


def _preprocess(edge_index, a_values):
    row = edge_index[0].astype(jnp.int32)
    col = edge_index[1].astype(jnp.int32)
    order = jnp.argsort(row)
    rs = row[order]
    cls = col[order]
    vs = a_values[order]

    t_e = rs // RPT
    cnt = jnp.bincount(t_e, length=TILES)
    nch = (cnt + CHUNK - 1) // CHUNK
    choff = jnp.concatenate([jnp.zeros((1,), jnp.int32),
                             jnp.cumsum(nch).astype(jnp.int32)])
    seg_start = choff * CHUNK
    cum_cnt = jnp.concatenate([jnp.zeros((1,), jnp.int32),
                               jnp.cumsum(cnt).astype(jnp.int32)])
    pos = seg_start[t_e] + (jnp.arange(E, dtype=jnp.int32) - cum_cnt[t_e])

    cols_p = jnp.zeros((EA,), jnp.int32).at[pos].set(cls)
    vals_p = jnp.zeros((EA,), jnp.float32).at[pos].set(vs)
    lrow_p = jnp.zeros((EA,), jnp.int32).at[pos].set(rs - t_e * RPT)

    meta = jnp.zeros((TILES, 16), jnp.int32)
    meta = meta.at[:, 0].set(choff[:-1])
    meta = meta.at[:, 1].set(choff[1:])

    cols2 = cols_p.reshape(EA // 128, 128)

    slot = jnp.arange(EA, dtype=jnp.int32)
    tile_of_slot = jnp.searchsorted(seg_start[1:], slot, side="right").astype(jnp.int32)
    grows = (10000 + tile_of_slot).at[pos].set(rs)
    lr640 = (640 + (tile_of_slot % 2)).at[pos].set(rs - (t_e // 2) * PROWS)
    nextrow = jnp.roll(grows, -1)
    lane15 = (slot % 16) == 15
    rowchg = grows != nextrow
    m1 = (rowchg | lane15).astype(jnp.int32)
    m2 = (rowchg & (~lane15)).astype(jnp.int32)
    rn640 = jnp.roll(lr640, -1)
    avals = jnp.abs(vals_p)
    pmeta = jnp.zeros((16, 16), jnp.int32)
    pmeta = pmeta.at[:, 0].set(choff[0:32:2]).at[:, 1].set(choff[2::2])
    v0 = jnp.concatenate([jnp.full((N,), 0.01, jnp.float32),
                          jnp.zeros((NP_ - N,), jnp.float32)])
    power_args = (cols_p, avals, lr640, rn640, m1, m2, pmeta, v0)
    return (cols2, vals_p, lrow_p, meta), power_args


def _spmm(y, pre):
    cols2, vals_p, lrow_p, meta = pre
    zeros = jnp.zeros((RPT, NH), jnp.float32)
    return _spmm_call(y, cols2, vals_p, lrow_p, meta, zeros)


def kernel(U, edge_index, A_values, W, B, V_w):
    pre, power_args = _preprocess(edge_index, A_values)

    w_out, v_out = _power_call(*power_args)
    rho = (jnp.linalg.norm(w_out)
           / jnp.maximum(jnp.linalg.norm(v_out), 1e-30)) + 1e-5
    kap_eff = KAPPA / rho
    kap_arr = jnp.full((8, NH), kap_eff, jnp.float32)
    Wp = _proj_call(W, kap_arr)

    u_pad = jnp.concatenate([U, jnp.zeros((NP_ - N, NH), jnp.float32)], axis=0)
    p_u = _spmm(u_pad, pre)
    aub, f0, d0 = _init_call(p_u, B)

    f_hist = jnp.zeros((5, NP_, NH), jnp.float32).at[0].set(f0)
    g_hist = jnp.zeros((5, NP_, NH), jnp.float32).at[0].set(f0)
    M = jnp.zeros((5, 5), jnp.float32).at[0, 0].set(jnp.sum(d0[0]))

    def step(alpha5, sl):
        a8 = jnp.zeros((8, NH), jnp.float32).at[:5, :].set(alpha5[:, None])
        xk, y = _prop_call(f_hist, a8, Wp)
        p = _spmm(y, pre)
        z, gn, d = _comb_call(p, aub, xk, g_hist)
        d6 = jnp.sum(d, axis=1)[:6]
        newrow = d6[:5].at[sl].set(d6[5])
        return z, gn, newrow

    alpha5 = jnp.zeros((5,), jnp.float32).at[0].set(1.0)
    z, gn, newrow = step(alpha5, 1)
    f_hist = f_hist.at[1].set(z)
    g_hist = g_hist.at[1].set(gn)
    M = M.at[1, :].set(newrow).at[:, 1].set(newrow)

    for k in range(2, THRESHOLD):
        nn = min(k, 5)
        sl = k % 5
        H = jnp.zeros((nn + 1, nn + 1), jnp.float32)
        H = H.at[0, 1:].set(1.0).at[1:, 0].set(1.0)
        H = H.at[1:, 1:].set(M[:nn, :nn] + LAM * jnp.eye(nn, dtype=jnp.float32))
        yv = jnp.zeros((nn + 1,), jnp.float32).at[0].set(1.0)
        alpha = jnp.linalg.solve(H, yv)[1:]
        alpha5 = jnp.zeros((5,), jnp.float32).at[:nn].set(alpha)
        z, gn, newrow = step(alpha5, sl)
        f_hist = f_hist.at[sl].set(z)
        g_hist = g_hist.at[sl].set(gn)
        M = M.at[sl, :].set(newrow).at[:, sl].set(newrow)

    z_star = f_hist[4]
    vwt = jnp.zeros((NH, NH), jnp.float32).at[:, :NCLASS].set(V_w.T)
    labels = _mm_call(z_star, vwt)
    return labels[:N, :NCLASS], z_star[:N]


# speedup vs baseline: 6.8990x; 1.5206x over previous
import functools

import jax
import jax.numpy as jnp
from jax import lax
from jax.experimental import pallas as pl
from jax.experimental.pallas import tpu as pltpu
from jax.experimental.pallas import tpu_sc as plsc

N = 10000
E = 160000
NH = 128
NCLASS = 16
KAPPA = 0.99
THRESHOLD = 20
LAM = 1e-4

NP_ = 10240
TILES = 32
RPT = NP_ // TILES
CHUNK = 256
EA = E + TILES * CHUNK
BLK = 512

@functools.cache
def _mesh():
    return plsc.VectorSubcoreMesh(
        core_axis_name="c", subcore_axis_name="s", num_cores=2, num_subcores=16
    )


HALF = 128


def _spmm_body(y_hbm, pack_hbm, meta_hbm, zeros_hbm,
               out_hbm, ebA, ebB, ybA, ybB, acc, mbuf, semGA, semGB):
    c = lax.axis_index("c")
    s = lax.axis_index("s")
    wid = s * 2 + c

    pltpu.sync_copy(meta_hbm.at[wid], mbuf)
    mv = mbuf[...]
    h_lo = mv[0] * 2
    h_hi = mv[1] * 2

    pltpu.sync_copy(zeros_hbm, acc)

    iota = lax.iota(jnp.int32, 16)

    def compute(eb, yb):
        def group_body(g, _):
            vv = plsc.bitcast(eb[1, pl.ds(g * 16, 16)], jnp.float32)
            lr = eb[2, pl.ds(g * 16, 16)]
            eids = g * 16 + iota
            for rot in range(16):
                lane = lax.bitwise_and(iota + rot, 15)
                for j in range(8):
                    x = plsc.load_gather(yb, [eids, j * 16 + lane])
                    plsc.addupdate_scatter(acc, [lr, j * 16 + lane], x * vv)
            return 0

        lax.fori_loop(0, HALF // 16, group_body, 0)

    def fetch(h, eb, yb, sem):
        pltpu.sync_copy(pack_hbm.at[h], eb)
        return pltpu.async_copy(y_hbm.at[eb.at[0]], yb, sem)

    @pl.when(h_lo < h_hi)
    def _():
        fetch(h_lo, ebA, ybA, semGA)

        @pl.when(h_lo + 1 < h_hi)
        def _():
            fetch(h_lo + 1, ebB, ybB, semGB)

        npairs = (h_hi - h_lo + 1) // 2

        def pair_body(i, _):
            h0 = h_lo + 2 * i
            h1 = h0 + 1
            pltpu.make_async_copy(y_hbm.at[pl.ds(0, HALF)], ybA, semGA).wait()
            compute(ebA, ybA)

            @pl.when(h0 + 2 < h_hi)
            def _():
                fetch(h0 + 2, ebA, ybA, semGA)

            @pl.when(h1 < h_hi)
            def _():
                pltpu.make_async_copy(
                    y_hbm.at[pl.ds(0, HALF)], ybB, semGB).wait()
                compute(ebB, ybB)

                @pl.when(h1 + 2 < h_hi)
                def _():
                    fetch(h1 + 2, ebB, ybB, semGB)
            return 0

        lax.fori_loop(0, npairs, pair_body, 0)

    pltpu.sync_copy(acc, out_hbm.at[pl.ds(wid * RPT, RPT)])


@functools.cache
def _spmm_kernel():
    return pl.kernel(
        _spmm_body,
        out_type=jax.ShapeDtypeStruct((NP_, NH), jnp.float32),
        mesh=_mesh(),
        scratch_types=[
            pltpu.VMEM((3, HALF), jnp.int32),
            pltpu.VMEM((3, HALF), jnp.int32),
            pltpu.VMEM((HALF, NH), jnp.float32),
            pltpu.VMEM((HALF, NH), jnp.float32),
            pltpu.VMEM((RPT, NH), jnp.float32),
            pltpu.VMEM((16,), jnp.int32),
            pltpu.SemaphoreType.DMA,
            pltpu.SemaphoreType.DMA,
        ],
        compiler_params=pltpu.CompilerParams(needs_layout_passes=False),
    )


def _spmm_call(*args):
    return _spmm_kernel()(*args)


PROWS = NP_ // 16
PCAP = 64
_POWER_ARG_SHAPES = [
    (((EA // CHUNK + PCAP) * 4 * CHUNK,), jnp.int32),
    ((16, 16), jnp.int32),
    ((NP_,), jnp.float32),
]


def _power_body(pack_hbm, pmeta_hbm, v0_hbm, wout_hbm, vout_hbm,
                epack, sbuf, vref, wref, vseg, maxb, mb2,
                mbuf, sh_s, sh_v, sem):
    c = lax.axis_index("c")
    sid = lax.axis_index("s")

    @pl.when(c == 0)
    def _():
        pltpu.sync_copy(pmeta_hbm.at[sid], mbuf)
        mv = mbuf[...]
        ch_lo = mv[0]
        ch_hi = mv[1]
        pltpu.sync_copy(v0_hbm, vref)
        pltpu.sync_copy(pack_hbm.at[pl.ds(ch_lo * (4 * CHUNK), PCAP * 4 * CHUNK)], epack)

        iota = lax.iota(jnp.int32, 16)
        not15 = iota != 15
        is15 = iota == 15

        def do_group(ref, base, g):
            cg = ref[pl.ds(base + g * 16, 16)]
            av = plsc.bitcast(ref[pl.ds(base + CHUNK + g * 16, 16)], jnp.float32)
            lr = ref[pl.ds(base + 2 * CHUNK + g * 16, 16)]
            rn = ref[pl.ds(base + 3 * CHUNK + g * 16, 16)]
            x = plsc.load_gather(vref, [cg])
            cs = plsc.cumsum(av * x)
            chg = lr != rn
            plsc.addupdate_scatter(wref, [lr], cs, mask=chg | is15)
            plsc.addupdate_scatter(wref, [rn], -cs, mask=chg & not15)

        def matvec():
            def zero_body(i, _):
                wref[pl.ds(i * 16, 16)] = jnp.zeros((16,), jnp.float32)
                return 0
            lax.fori_loop(0, (PROWS + 16) // 16, zero_body, 0)

            def chunk_body(ci, _):
                @pl.when(ci < PCAP)
                def _():
                    def group_body(g2, _):
                        do_group(epack, ci * (4 * CHUNK), 2 * g2)
                        do_group(epack, ci * (4 * CHUNK), 2 * g2 + 1)
                        return 0
                    lax.fori_loop(0, CHUNK // 32, group_body, 0)

                @pl.when(ci >= PCAP)
                def _():
                    pltpu.sync_copy(
                        pack_hbm.at[pl.ds((ch_lo + ci) * (4 * CHUNK),
                                          4 * CHUNK)], sbuf)

                    def group_body(g2, _):
                        do_group(sbuf, 0, 2 * g2)
                        do_group(sbuf, 0, 2 * g2 + 1)
                        return 0
                    lax.fori_loop(0, CHUNK // 32, group_body, 0)
                return 0

            lax.fori_loop(0, ch_hi - ch_lo, chunk_body, 0)

        def iter_body(_, __):
            matvec()
            def max_body(i, m):
                return jnp.maximum(m, jnp.abs(wref[pl.ds(i * 16, 16)]))
            m = lax.fori_loop(0, PROWS // 16, max_body,
                              jnp.zeros((16,), jnp.float32))
            maxb[...] = jnp.full((16,), jnp.max(m), jnp.float32)
            pltpu.sync_copy(maxb, sh_s.at[sid])
            plsc.subcore_barrier()
            pltpu.sync_copy(sh_s, mb2)

            def gmax_body(t, m):
                return jnp.maximum(m, mb2[t])
            gm = lax.fori_loop(0, 16, gmax_body, jnp.zeros((16,), jnp.float32))
            sv = jnp.full((16,), jnp.max(gm), jnp.float32)
            invv = jnp.full((16,), 1.0, jnp.float32) / jnp.maximum(sv, 1e-30)

            def scale_body(i, _):
                vseg[pl.ds(i * 16, 16)] = wref[pl.ds(i * 16, 16)] * invv
                return 0
            lax.fori_loop(0, PROWS // 16, scale_body, 0)
            pltpu.sync_copy(vseg, sh_v.at[pl.ds(sid * PROWS, PROWS)])
            plsc.subcore_barrier()
            pltpu.sync_copy(sh_v, vref)
            plsc.subcore_barrier()
            return 0

        lax.fori_loop(0, 50, iter_body, 0)
        matvec()

        def out_body(i, _):
            vseg[pl.ds(i * 16, 16)] = wref[pl.ds(i * 16, 16)]
            return 0
        lax.fori_loop(0, PROWS // 16, out_body, 0)
        pltpu.sync_copy(vseg, wout_hbm.at[pl.ds(sid * PROWS, PROWS)])
        pltpu.sync_copy(vref.at[pl.ds(sid * PROWS, PROWS)],
                        vout_hbm.at[pl.ds(sid * PROWS, PROWS)])


@functools.cache
def _power_kernel():
    return pl.kernel(
        _power_body,
        out_type=(
            jax.ShapeDtypeStruct((NP_,), jnp.float32),
            jax.ShapeDtypeStruct((NP_,), jnp.float32),
        ),
        mesh=_mesh(),
        scratch_types=[
            pltpu.VMEM((PCAP * 4 * CHUNK,), jnp.int32),
            pltpu.VMEM((4 * CHUNK,), jnp.int32),
            pltpu.VMEM((NP_,), jnp.float32),
            pltpu.VMEM((PROWS + 16,), jnp.float32),
            pltpu.VMEM((PROWS,), jnp.float32),
            pltpu.VMEM((16,), jnp.float32),
            pltpu.VMEM((16, 16), jnp.float32),
            pltpu.VMEM((16,), jnp.int32),
            pltpu.VMEM_SHARED((16, 16), jnp.float32),
            pltpu.VMEM_SHARED((NP_,), jnp.float32),
            pltpu.SemaphoreType.DMA,
        ],
        compiler_params=pltpu.CompilerParams(needs_layout_passes=False),
    )


def _power_call(*args):
    return _power_kernel()(*args)


def _prop_body(f_ref, a_ref, w_ref, xk_ref, y_ref):
    F = f_ref[...]
    al = a_ref[...]
    xk = jnp.sum(F * al[:5][:, None, :], axis=0)
    xk_ref[...] = xk
    y_ref[...] = jnp.dot(xk, w_ref[...], preferred_element_type=jnp.float32)


def _prop_call(f_hist, a8, wp):
    return pl.pallas_call(
        _prop_body,
        grid=(NP_ // BLK,),
        in_specs=[
            pl.BlockSpec((5, BLK, NH), lambda i: (0, i, 0)),
            pl.BlockSpec((8, NH), lambda i: (0, 0)),
            pl.BlockSpec((NH, NH), lambda i: (0, 0)),
        ],
        out_specs=[
            pl.BlockSpec((BLK, NH), lambda i: (i, 0)),
            pl.BlockSpec((BLK, NH), lambda i: (i, 0)),
        ],
        out_shape=[
            jax.ShapeDtypeStruct((NP_, NH), jnp.float32),
            jax.ShapeDtypeStruct((NP_, NH), jnp.float32),
        ],
    )(f_hist, a8, wp)


def _comb_body(p_ref, aub_ref, xk_ref, g_ref, z_ref, gn_ref, d_ref):
    z = jnp.maximum(p_ref[...] + aub_ref[...], 0.0)
    gnew = z - xk_ref[...]
    z_ref[...] = z
    gn_ref[...] = gnew
    G = g_ref[...]
    parts = jnp.sum(G * gnew[None], axis=1)
    selfp = jnp.sum(gnew * gnew, axis=0)[None]
    d = jnp.concatenate([parts, selfp, jnp.zeros((2, NH), jnp.float32)], axis=0)

    @pl.when(pl.program_id(0) == 0)
    def _():
        d_ref[...] = jnp.zeros_like(d_ref)

    d_ref[...] += d


def _comb_call(p, aub, xk, g_hist):
    return pl.pallas_call(
        _comb_body,
        grid=(NP_ // BLK,),
        in_specs=[
            pl.BlockSpec((BLK, NH), lambda i: (i, 0)),
            pl.BlockSpec((BLK, NH), lambda i: (i, 0)),
            pl.BlockSpec((BLK, NH), lambda i: (i, 0)),
            pl.BlockSpec((5, BLK, NH), lambda i: (0, i, 0)),
        ],
        out_specs=[
            pl.BlockSpec((BLK, NH), lambda i: (i, 0)),
            pl.BlockSpec((BLK, NH), lambda i: (i, 0)),
            pl.BlockSpec((8, NH), lambda i: (0, 0)),
        ],
        out_shape=[
            jax.ShapeDtypeStruct((NP_, NH), jnp.float32),
            jax.ShapeDtypeStruct((NP_, NH), jnp.float32),
            jax.ShapeDtypeStruct((8, NH), jnp.float32),
        ],
    )(p, aub, xk, g_hist)


def _init_body(pu_ref, b_ref, aub_ref, f0_ref, d_ref):
    aub = jnp.dot(pu_ref[...], b_ref[...], preferred_element_type=jnp.float32)
    f0 = jnp.maximum(aub, 0.0)
    aub_ref[...] = aub
    f0_ref[...] = f0
    d = jnp.concatenate(
        [jnp.sum(f0 * f0, axis=0)[None], jnp.zeros((7, NH), jnp.float32)], axis=0
    )

    @pl.when(pl.program_id(0) == 0)
    def _():
        d_ref[...] = jnp.zeros_like(d_ref)

    d_ref[...] += d


def _init_call(pu, b):
    return pl.pallas_call(
        _init_body,
        grid=(NP_ // BLK,),
        in_specs=[
            pl.BlockSpec((BLK, NH), lambda i: (i, 0)),
            pl.BlockSpec((NH, NH), lambda i: (0, 0)),
        ],
        out_specs=[
            pl.BlockSpec((BLK, NH), lambda i: (i, 0)),
            pl.BlockSpec((BLK, NH), lambda i: (i, 0)),
            pl.BlockSpec((8, NH), lambda i: (0, 0)),
        ],
        out_shape=[
            jax.ShapeDtypeStruct((NP_, NH), jnp.float32),
            jax.ShapeDtypeStruct((NP_, NH), jnp.float32),
            jax.ShapeDtypeStruct((8, NH), jnp.float32),
        ],
    )(pu, b)


def _proj_body(w_ref, kap_ref, out_ref):
    W = w_ref[...]
    kapc = kap_ref[...][0:1, 0:1]
    a = jnp.abs(W)
    s = jnp.sum(a, axis=1, keepdims=True)
    hi0 = jnp.max(a, axis=1, keepdims=True)

    def bis(_, lh):
        lo, hi = lh
        mid = 0.5 * (lo + hi)
        t = jnp.sum(jnp.maximum(a - mid, 0.0), axis=1, keepdims=True)
        pred = t > kapc
        return jnp.where(pred, mid, lo), jnp.where(pred, hi, mid)

    lo, hi = lax.fori_loop(0, 60, bis, (jnp.zeros_like(s), hi0))
    theta = 0.5 * (lo + hi)
    proj = jnp.sign(W) * jnp.maximum(a - theta, 0.0)
    out_ref[...] = jnp.where(s > kapc, proj, W)


def _proj_call(w, kap):
    return pl.pallas_call(
        _proj_body,
        in_specs=[
            pl.BlockSpec((NH, NH), lambda: (0, 0)),
            pl.BlockSpec((8, NH), lambda: (0, 0)),
        ],
        out_specs=pl.BlockSpec((NH, NH), lambda: (0, 0)),
        out_shape=jax.ShapeDtypeStruct((NH, NH), jnp.float32),
    )(w, kap)


def _mm_body(x_ref, w_ref, o_ref):
    o_ref[...] = jnp.dot(x_ref[...], w_ref[...], preferred_element_type=jnp.float32)


def _mm_call(x, w):
    return pl.pallas_call(
        _mm_body,
        grid=(NP_ // BLK,),
        in_specs=[
            pl.BlockSpec((BLK, NH), lambda i: (i, 0)),
            pl.BlockSpec((NH, NH), lambda i: (0, 0)),
        ],
        out_specs=pl.BlockSpec((BLK, NH), lambda i: (i, 0)),
        out_shape=jax.ShapeDtypeStruct((NP_, NH), jnp.float32),
    )(x, w)


def _preprocess(edge_index, a_values):
    row = edge_index[0].astype(jnp.int32)
    col = edge_index[1].astype(jnp.int32)
    order = jnp.argsort(row)
    rs = row[order]
    cls = col[order]
    vs = a_values[order]

    t_e = rs // RPT
    cnt = jnp.bincount(t_e, length=TILES)
    nch = (cnt + CHUNK - 1) // CHUNK
    choff = jnp.concatenate([jnp.zeros((1,), jnp.int32),
                             jnp.cumsum(nch).astype(jnp.int32)])
    seg_start = choff * CHUNK
    cum_cnt = jnp.concatenate([jnp.zeros((1,), jnp.int32),
                               jnp.cumsum(cnt).astype(jnp.int32)])
    pos = seg_start[t_e] + (jnp.arange(E, dtype=jnp.int32) - cum_cnt[t_e])

    cols_p = jnp.zeros((EA,), jnp.int32).at[pos].set(cls)
    vals_p = jnp.zeros((EA,), jnp.float32).at[pos].set(vs)
    lrow_p = jnp.zeros((EA,), jnp.int32).at[pos].set(rs - t_e * RPT)

    meta = jnp.zeros((TILES, 16), jnp.int32)
    meta = meta.at[:, 0].set(choff[:-1])
    meta = meta.at[:, 1].set(choff[1:])

    vbits = lax.bitcast_convert_type(vals_p, jnp.int32)
    pack_spmm = jnp.stack(
        [cols_p.reshape(EA // HALF, HALF),
         vbits.reshape(EA // HALF, HALF),
         lrow_p.reshape(EA // HALF, HALF)], axis=1)

    slot = jnp.arange(EA, dtype=jnp.int32)
    tile_of_slot = jnp.searchsorted(seg_start[1:], slot, side="right").astype(jnp.int32)
    lr640 = (640 + (tile_of_slot % 2)).at[pos].set(rs - (t_e // 2) * PROWS)
    rn640 = jnp.roll(lr640, -1)
    abits = lax.bitcast_convert_type(jnp.abs(vals_p), jnp.int32)
    pack_pow = jnp.concatenate(
        [cols_p.reshape(EA // CHUNK, CHUNK),
         abits.reshape(EA // CHUNK, CHUNK),
         lr640.reshape(EA // CHUNK, CHUNK),
         rn640.reshape(EA // CHUNK, CHUNK)], axis=1)
    pack_pow = jnp.concatenate(
        [pack_pow, jnp.zeros((PCAP, 4 * CHUNK), jnp.int32)], axis=0).reshape(-1)
    pmeta = jnp.zeros((16, 16), jnp.int32)
    pmeta = pmeta.at[:, 0].set(choff[0:32:2]).at[:, 1].set(choff[2::2])
    v0 = jnp.concatenate([jnp.full((N,), 0.01, jnp.float32),
                          jnp.zeros((NP_ - N,), jnp.float32)])
    power_args = (pack_pow, pmeta, v0)
    return (pack_spmm, meta), power_args


def _spmm(y, pre):
    pack_spmm, meta = pre
    zeros = jnp.zeros((RPT, NH), jnp.float32)
    return _spmm_call(y, pack_spmm, meta, zeros)


def kernel(U, edge_index, A_values, W, B, V_w):
    pre, power_args = _preprocess(edge_index, A_values)

    w_out, v_out = _power_call(*power_args)
    rho = (jnp.linalg.norm(w_out)
           / jnp.maximum(jnp.linalg.norm(v_out), 1e-30)) + 1e-5
    kap_eff = KAPPA / rho
    kap_arr = jnp.full((8, NH), kap_eff, jnp.float32)
    Wp = _proj_call(W, kap_arr)

    u_pad = jnp.concatenate([U, jnp.zeros((NP_ - N, NH), jnp.float32)], axis=0)
    p_u = _spmm(u_pad, pre)
    aub, f0, d0 = _init_call(p_u, B)

    f_hist = jnp.zeros((5, NP_, NH), jnp.float32).at[0].set(f0)
    g_hist = jnp.zeros((5, NP_, NH), jnp.float32).at[0].set(f0)
    M = jnp.zeros((5, 5), jnp.float32).at[0, 0].set(jnp.sum(d0[0]))

    def step(alpha5, sl):
        a8 = jnp.zeros((8, NH), jnp.float32).at[:5, :].set(alpha5[:, None])
        xk, y = _prop_call(f_hist, a8, Wp)
        p = _spmm(y, pre)
        z, gn, d = _comb_call(p, aub, xk, g_hist)
        d6 = jnp.sum(d, axis=1)[:6]
        newrow = d6[:5].at[sl].set(d6[5])
        return z, gn, newrow

    alpha5 = jnp.zeros((5,), jnp.float32).at[0].set(1.0)
    z, gn, newrow = step(alpha5, 1)
    f_hist = f_hist.at[1].set(z)
    g_hist = g_hist.at[1].set(gn)
    M = M.at[1, :].set(newrow).at[:, 1].set(newrow)

    for k in range(2, THRESHOLD):
        nn = min(k, 5)
        sl = k % 5
        H = jnp.zeros((nn + 1, nn + 1), jnp.float32)
        H = H.at[0, 1:].set(1.0).at[1:, 0].set(1.0)
        H = H.at[1:, 1:].set(M[:nn, :nn] + LAM * jnp.eye(nn, dtype=jnp.float32))
        yv = jnp.zeros((nn + 1,), jnp.float32).at[0].set(1.0)
        alpha = jnp.linalg.solve(H, yv)[1:]
        alpha5 = jnp.zeros((5,), jnp.float32).at[:nn].set(alpha)
        z, gn, newrow = step(alpha5, sl)
        f_hist = f_hist.at[sl].set(z)
        g_hist = g_hist.at[sl].set(gn)
        M = M.at[sl, :].set(newrow).at[:, sl].set(newrow)

    z_star = f_hist[4]
    vwt = jnp.zeros((NH, NH), jnp.float32).at[:, :NCLASS].set(V_w.T)
    labels = _mm_call(z_star, vwt)
    return labels[:N, :NCLASS], z_star[:N]


# speedup vs baseline: 7.2559x; 1.0517x over previous
import functools

import jax
import jax.numpy as jnp
from jax import lax
from jax.experimental import pallas as pl
from jax.experimental.pallas import tpu as pltpu
from jax.experimental.pallas import tpu_sc as plsc

N = 10000
E = 160000
NH = 128
NCLASS = 16
KAPPA = 0.99
THRESHOLD = 20
LAM = 1e-4

NP_ = 10240
TILES = 32
RPT = NP_ // TILES
CHUNK = 256
EA = E + TILES * CHUNK
BLK = 512

@functools.cache
def _mesh():
    return plsc.VectorSubcoreMesh(
        core_axis_name="c", subcore_axis_name="s", num_cores=2, num_subcores=16
    )


HALF = 128


def _spmm_body(y_hbm, pack_hbm, meta_hbm, zeros_hbm,
               out_hbm, ebA, ebB, ebC, ebD, ybA, ybB, ybC, ybD, acc, mbuf,
               semGA, semGB, semGC, semGD):
    c = lax.axis_index("c")
    s = lax.axis_index("s")
    wid = s * 2 + c

    pltpu.sync_copy(meta_hbm.at[wid], mbuf)
    mv = mbuf[...]
    h_lo = mv[0] * 2
    h_hi = mv[1] * 2

    pltpu.sync_copy(zeros_hbm, acc)

    iota = lax.iota(jnp.int32, 16)

    def compute(eb, yb):
        def group_body(g, _):
            vv = plsc.bitcast(eb[1, pl.ds(g * 16, 16)], jnp.float32)
            lr = eb[2, pl.ds(g * 16, 16)]
            eids = g * 16 + iota
            for rot in range(16):
                lane = lax.bitwise_and(iota + rot, 15)
                for j in range(8):
                    x = plsc.load_gather(yb, [eids, j * 16 + lane])
                    plsc.addupdate_scatter(acc, [lr, j * 16 + lane], x * vv)
            return 0

        lax.fori_loop(0, HALF // 16, group_body, 0)

    def fetch(h, eb, yb, sem):
        pltpu.sync_copy(pack_hbm.at[h], eb)
        return pltpu.async_copy(y_hbm.at[eb.at[0]], yb, sem)

    ebs = [ebA, ebB, ebC, ebD]
    ybs = [ybA, ybB, ybC, ybD]
    sems = [semGA, semGB, semGC, semGD]

    @pl.when(h_lo < h_hi)
    def _():
        for b in range(4):
            if b == 0:
                fetch(h_lo, ebs[0], ybs[0], sems[0])
            else:
                @pl.when(h_lo + b < h_hi)
                def _(b=b):
                    fetch(h_lo + b, ebs[b], ybs[b], sems[b])

        nquads = (h_hi - h_lo + 3) // 4

        def quad_body(i, _):
            for b in range(4):
                h = h_lo + 4 * i + b

                def phase(b=b, h=h):
                    pltpu.make_async_copy(
                        y_hbm.at[pl.ds(0, HALF)], ybs[b], sems[b]).wait()
                    compute(ebs[b], ybs[b])

                    @pl.when(h + 4 < h_hi)
                    def _():
                        fetch(h + 4, ebs[b], ybs[b], sems[b])

                if b == 0:
                    phase()
                else:
                    pl.when(h < h_hi)(phase)
            return 0

        lax.fori_loop(0, nquads, quad_body, 0)

    pltpu.sync_copy(acc, out_hbm.at[pl.ds(wid * RPT, RPT)])


@functools.cache
def _spmm_kernel():
    return pl.kernel(
        _spmm_body,
        out_type=jax.ShapeDtypeStruct((NP_, NH), jnp.float32),
        mesh=_mesh(),
        scratch_types=[
            pltpu.VMEM((3, HALF), jnp.int32),
            pltpu.VMEM((3, HALF), jnp.int32),
            pltpu.VMEM((3, HALF), jnp.int32),
            pltpu.VMEM((3, HALF), jnp.int32),
            pltpu.VMEM((HALF, NH), jnp.float32),
            pltpu.VMEM((HALF, NH), jnp.float32),
            pltpu.VMEM((HALF, NH), jnp.float32),
            pltpu.VMEM((HALF, NH), jnp.float32),
            pltpu.VMEM((RPT, NH), jnp.float32),
            pltpu.VMEM((16,), jnp.int32),
            pltpu.SemaphoreType.DMA,
            pltpu.SemaphoreType.DMA,
            pltpu.SemaphoreType.DMA,
            pltpu.SemaphoreType.DMA,
        ],
        compiler_params=pltpu.CompilerParams(needs_layout_passes=False),
    )


def _spmm_call(*args):
    return _spmm_kernel()(*args)


PROWS = NP_ // 16
PCAP = 64
_POWER_ARG_SHAPES = [
    (((EA // CHUNK + PCAP) * 4 * CHUNK,), jnp.int32),
    ((16, 16), jnp.int32),
    ((NP_,), jnp.float32),
]


def _power_body(pack_hbm, pmeta_hbm, v0_hbm, wout_hbm, vout_hbm,
                epack, sbuf, vref, wref, vseg, maxb, mb2,
                mbuf, sh_s, sh_v, sem):
    c = lax.axis_index("c")
    sid = lax.axis_index("s")

    @pl.when(c == 0)
    def _():
        pltpu.sync_copy(pmeta_hbm.at[sid], mbuf)
        mv = mbuf[...]
        ch_lo = mv[0]
        ch_hi = mv[1]
        pltpu.sync_copy(v0_hbm, vref)
        pltpu.sync_copy(pack_hbm.at[pl.ds(ch_lo * (4 * CHUNK), PCAP * 4 * CHUNK)], epack)

        iota = lax.iota(jnp.int32, 16)
        not15 = iota != 15
        is15 = iota == 15

        def do_group(ref, base, g):
            cg = ref[pl.ds(base + g * 16, 16)]
            av = plsc.bitcast(ref[pl.ds(base + CHUNK + g * 16, 16)], jnp.float32)
            lr = ref[pl.ds(base + 2 * CHUNK + g * 16, 16)]
            rn = ref[pl.ds(base + 3 * CHUNK + g * 16, 16)]
            x = plsc.load_gather(vref, [cg])
            cs = plsc.cumsum(av * x)
            chg = lr != rn
            plsc.addupdate_scatter(wref, [lr], cs, mask=chg | is15)
            plsc.addupdate_scatter(wref, [rn], -cs, mask=chg & not15)

        def matvec():
            def zero_body(i, _):
                wref[pl.ds(i * 16, 16)] = jnp.zeros((16,), jnp.float32)
                return 0
            lax.fori_loop(0, (PROWS + 16) // 16, zero_body, 0)

            def chunk_body(ci, _):
                @pl.when(ci < PCAP)
                def _():
                    def group_body(g2, _):
                        do_group(epack, ci * (4 * CHUNK), 2 * g2)
                        do_group(epack, ci * (4 * CHUNK), 2 * g2 + 1)
                        return 0
                    lax.fori_loop(0, CHUNK // 32, group_body, 0)

                @pl.when(ci >= PCAP)
                def _():
                    pltpu.sync_copy(
                        pack_hbm.at[pl.ds((ch_lo + ci) * (4 * CHUNK),
                                          4 * CHUNK)], sbuf)

                    def group_body(g2, _):
                        do_group(sbuf, 0, 2 * g2)
                        do_group(sbuf, 0, 2 * g2 + 1)
                        return 0
                    lax.fori_loop(0, CHUNK // 32, group_body, 0)
                return 0

            lax.fori_loop(0, ch_hi - ch_lo, chunk_body, 0)

        def iter_body(_, __):
            matvec()
            def max_body(i, m):
                return jnp.maximum(m, jnp.abs(wref[pl.ds(i * 16, 16)]))
            m = lax.fori_loop(0, PROWS // 16, max_body,
                              jnp.zeros((16,), jnp.float32))
            maxb[...] = jnp.full((16,), jnp.max(m), jnp.float32)
            pltpu.sync_copy(maxb, sh_s.at[sid])
            plsc.subcore_barrier()
            pltpu.sync_copy(sh_s, mb2)

            def gmax_body(t, m):
                return jnp.maximum(m, mb2[t])
            gm = lax.fori_loop(0, 16, gmax_body, jnp.zeros((16,), jnp.float32))
            sv = jnp.full((16,), jnp.max(gm), jnp.float32)
            invv = jnp.full((16,), 1.0, jnp.float32) / jnp.maximum(sv, 1e-30)

            def scale_body(i, _):
                vseg[pl.ds(i * 16, 16)] = wref[pl.ds(i * 16, 16)] * invv
                return 0
            lax.fori_loop(0, PROWS // 16, scale_body, 0)
            pltpu.sync_copy(vseg, sh_v.at[pl.ds(sid * PROWS, PROWS)])
            plsc.subcore_barrier()
            pltpu.sync_copy(sh_v, vref)
            plsc.subcore_barrier()
            return 0

        lax.fori_loop(0, 50, iter_body, 0)
        matvec()

        def out_body(i, _):
            vseg[pl.ds(i * 16, 16)] = wref[pl.ds(i * 16, 16)]
            return 0
        lax.fori_loop(0, PROWS // 16, out_body, 0)
        pltpu.sync_copy(vseg, wout_hbm.at[pl.ds(sid * PROWS, PROWS)])
        pltpu.sync_copy(vref.at[pl.ds(sid * PROWS, PROWS)],
                        vout_hbm.at[pl.ds(sid * PROWS, PROWS)])


@functools.cache
def _power_kernel():
    return pl.kernel(
        _power_body,
        out_type=(
            jax.ShapeDtypeStruct((NP_,), jnp.float32),
            jax.ShapeDtypeStruct((NP_,), jnp.float32),
        ),
        mesh=_mesh(),
        scratch_types=[
            pltpu.VMEM((PCAP * 4 * CHUNK,), jnp.int32),
            pltpu.VMEM((4 * CHUNK,), jnp.int32),
            pltpu.VMEM((NP_,), jnp.float32),
            pltpu.VMEM((PROWS + 16,), jnp.float32),
            pltpu.VMEM((PROWS,), jnp.float32),
            pltpu.VMEM((16,), jnp.float32),
            pltpu.VMEM((16, 16), jnp.float32),
            pltpu.VMEM((16,), jnp.int32),
            pltpu.VMEM_SHARED((16, 16), jnp.float32),
            pltpu.VMEM_SHARED((NP_,), jnp.float32),
            pltpu.SemaphoreType.DMA,
        ],
        compiler_params=pltpu.CompilerParams(needs_layout_passes=False),
    )


def _power_call(*args):
    return _power_kernel()(*args)


def _prop_body(f_ref, a_ref, w_ref, xk_ref, y_ref):
    F = f_ref[...]
    al = a_ref[...]
    xk = jnp.sum(F * al[:5][:, None, :], axis=0)
    xk_ref[...] = xk
    y_ref[...] = jnp.dot(xk, w_ref[...], preferred_element_type=jnp.float32)


def _prop_call(f_hist, a8, wp):
    return pl.pallas_call(
        _prop_body,
        grid=(NP_ // BLK,),
        in_specs=[
            pl.BlockSpec((5, BLK, NH), lambda i: (0, i, 0)),
            pl.BlockSpec((8, NH), lambda i: (0, 0)),
            pl.BlockSpec((NH, NH), lambda i: (0, 0)),
        ],
        out_specs=[
            pl.BlockSpec((BLK, NH), lambda i: (i, 0)),
            pl.BlockSpec((BLK, NH), lambda i: (i, 0)),
        ],
        out_shape=[
            jax.ShapeDtypeStruct((NP_, NH), jnp.float32),
            jax.ShapeDtypeStruct((NP_, NH), jnp.float32),
        ],
    )(f_hist, a8, wp)


def _comb_body(p_ref, aub_ref, xk_ref, g_ref, z_ref, gn_ref, d_ref):
    z = jnp.maximum(p_ref[...] + aub_ref[...], 0.0)
    gnew = z - xk_ref[...]
    z_ref[...] = z
    gn_ref[...] = gnew
    G = g_ref[...]
    parts = jnp.sum(G * gnew[None], axis=1)
    selfp = jnp.sum(gnew * gnew, axis=0)[None]
    d = jnp.concatenate([parts, selfp, jnp.zeros((2, NH), jnp.float32)], axis=0)

    @pl.when(pl.program_id(0) == 0)
    def _():
        d_ref[...] = jnp.zeros_like(d_ref)

    d_ref[...] += d


def _comb_call(p, aub, xk, g_hist):
    return pl.pallas_call(
        _comb_body,
        grid=(NP_ // BLK,),
        in_specs=[
            pl.BlockSpec((BLK, NH), lambda i: (i, 0)),
            pl.BlockSpec((BLK, NH), lambda i: (i, 0)),
            pl.BlockSpec((BLK, NH), lambda i: (i, 0)),
            pl.BlockSpec((5, BLK, NH), lambda i: (0, i, 0)),
        ],
        out_specs=[
            pl.BlockSpec((BLK, NH), lambda i: (i, 0)),
            pl.BlockSpec((BLK, NH), lambda i: (i, 0)),
            pl.BlockSpec((8, NH), lambda i: (0, 0)),
        ],
        out_shape=[
            jax.ShapeDtypeStruct((NP_, NH), jnp.float32),
            jax.ShapeDtypeStruct((NP_, NH), jnp.float32),
            jax.ShapeDtypeStruct((8, NH), jnp.float32),
        ],
    )(p, aub, xk, g_hist)


def _init_body(pu_ref, b_ref, aub_ref, f0_ref, d_ref):
    aub = jnp.dot(pu_ref[...], b_ref[...], preferred_element_type=jnp.float32)
    f0 = jnp.maximum(aub, 0.0)
    aub_ref[...] = aub
    f0_ref[...] = f0
    d = jnp.concatenate(
        [jnp.sum(f0 * f0, axis=0)[None], jnp.zeros((7, NH), jnp.float32)], axis=0
    )

    @pl.when(pl.program_id(0) == 0)
    def _():
        d_ref[...] = jnp.zeros_like(d_ref)

    d_ref[...] += d


def _init_call(pu, b):
    return pl.pallas_call(
        _init_body,
        grid=(NP_ // BLK,),
        in_specs=[
            pl.BlockSpec((BLK, NH), lambda i: (i, 0)),
            pl.BlockSpec((NH, NH), lambda i: (0, 0)),
        ],
        out_specs=[
            pl.BlockSpec((BLK, NH), lambda i: (i, 0)),
            pl.BlockSpec((BLK, NH), lambda i: (i, 0)),
            pl.BlockSpec((8, NH), lambda i: (0, 0)),
        ],
        out_shape=[
            jax.ShapeDtypeStruct((NP_, NH), jnp.float32),
            jax.ShapeDtypeStruct((NP_, NH), jnp.float32),
            jax.ShapeDtypeStruct((8, NH), jnp.float32),
        ],
    )(pu, b)


def _proj_body(w_ref, kap_ref, out_ref):
    W = w_ref[...]
    kapc = kap_ref[...][0:1, 0:1]
    a = jnp.abs(W)
    s = jnp.sum(a, axis=1, keepdims=True)
    hi0 = jnp.max(a, axis=1, keepdims=True)

    def bis(_, lh):
        lo, hi = lh
        mid = 0.5 * (lo + hi)
        t = jnp.sum(jnp.maximum(a - mid, 0.0), axis=1, keepdims=True)
        pred = t > kapc
        return jnp.where(pred, mid, lo), jnp.where(pred, hi, mid)

    lo, hi = lax.fori_loop(0, 60, bis, (jnp.zeros_like(s), hi0))
    theta = 0.5 * (lo + hi)
    proj = jnp.sign(W) * jnp.maximum(a - theta, 0.0)
    out_ref[...] = jnp.where(s > kapc, proj, W)


def _proj_call(w, kap):
    return pl.pallas_call(
        _proj_body,
        in_specs=[
            pl.BlockSpec((NH, NH), lambda: (0, 0)),
            pl.BlockSpec((8, NH), lambda: (0, 0)),
        ],
        out_specs=pl.BlockSpec((NH, NH), lambda: (0, 0)),
        out_shape=jax.ShapeDtypeStruct((NH, NH), jnp.float32),
    )(w, kap)


def _mm_body(x_ref, w_ref, o_ref):
    o_ref[...] = jnp.dot(x_ref[...], w_ref[...], preferred_element_type=jnp.float32)


def _mm_call(x, w):
    return pl.pallas_call(
        _mm_body,
        grid=(NP_ // BLK,),
        in_specs=[
            pl.BlockSpec((BLK, NH), lambda i: (i, 0)),
            pl.BlockSpec((NH, NH), lambda i: (0, 0)),
        ],
        out_specs=pl.BlockSpec((BLK, NH), lambda i: (i, 0)),
        out_shape=jax.ShapeDtypeStruct((NP_, NH), jnp.float32),
    )(x, w)


def _preprocess(edge_index, a_values):
    row = edge_index[0].astype(jnp.int32)
    col = edge_index[1].astype(jnp.int32)
    order = jnp.argsort(row)
    rs = row[order]
    cls = col[order]
    vs = a_values[order]

    t_e = rs // RPT
    cnt = jnp.bincount(t_e, length=TILES)
    nch = (cnt + CHUNK - 1) // CHUNK
    choff = jnp.concatenate([jnp.zeros((1,), jnp.int32),
                             jnp.cumsum(nch).astype(jnp.int32)])
    seg_start = choff * CHUNK
    cum_cnt = jnp.concatenate([jnp.zeros((1,), jnp.int32),
                               jnp.cumsum(cnt).astype(jnp.int32)])
    pos = seg_start[t_e] + (jnp.arange(E, dtype=jnp.int32) - cum_cnt[t_e])

    cols_p = jnp.zeros((EA,), jnp.int32).at[pos].set(cls)
    vals_p = jnp.zeros((EA,), jnp.float32).at[pos].set(vs)
    lrow_p = jnp.zeros((EA,), jnp.int32).at[pos].set(rs - t_e * RPT)

    meta = jnp.zeros((TILES, 16), jnp.int32)
    meta = meta.at[:, 0].set(choff[:-1])
    meta = meta.at[:, 1].set(choff[1:])

    vbits = lax.bitcast_convert_type(vals_p, jnp.int32)
    pack_spmm = jnp.stack(
        [cols_p.reshape(EA // HALF, HALF),
         vbits.reshape(EA // HALF, HALF),
         lrow_p.reshape(EA // HALF, HALF)], axis=1)

    slot = jnp.arange(EA, dtype=jnp.int32)
    tile_of_slot = jnp.searchsorted(seg_start[1:], slot, side="right").astype(jnp.int32)
    lr640 = (640 + (tile_of_slot % 2)).at[pos].set(rs - (t_e // 2) * PROWS)
    rn640 = jnp.roll(lr640, -1)
    abits = lax.bitcast_convert_type(jnp.abs(vals_p), jnp.int32)
    pack_pow = jnp.concatenate(
        [cols_p.reshape(EA // CHUNK, CHUNK),
         abits.reshape(EA // CHUNK, CHUNK),
         lr640.reshape(EA // CHUNK, CHUNK),
         rn640.reshape(EA // CHUNK, CHUNK)], axis=1)
    pack_pow = jnp.concatenate(
        [pack_pow, jnp.zeros((PCAP, 4 * CHUNK), jnp.int32)], axis=0).reshape(-1)
    pmeta = jnp.zeros((16, 16), jnp.int32)
    pmeta = pmeta.at[:, 0].set(choff[0:32:2]).at[:, 1].set(choff[2::2])
    v0 = jnp.concatenate([jnp.full((N,), 0.01, jnp.float32),
                          jnp.zeros((NP_ - N,), jnp.float32)])
    power_args = (pack_pow, pmeta, v0)
    return (pack_spmm, meta), power_args


def _spmm(y, pre):
    pack_spmm, meta = pre
    zeros = jnp.zeros((RPT, NH), jnp.float32)
    return _spmm_call(y, pack_spmm, meta, zeros)


def kernel(U, edge_index, A_values, W, B, V_w):
    pre, power_args = _preprocess(edge_index, A_values)

    w_out, v_out = _power_call(*power_args)
    rho = (jnp.linalg.norm(w_out)
           / jnp.maximum(jnp.linalg.norm(v_out), 1e-30)) + 1e-5
    kap_eff = KAPPA / rho
    kap_arr = jnp.full((8, NH), kap_eff, jnp.float32)
    Wp = _proj_call(W, kap_arr)

    u_pad = jnp.concatenate([U, jnp.zeros((NP_ - N, NH), jnp.float32)], axis=0)
    p_u = _spmm(u_pad, pre)
    aub, f0, d0 = _init_call(p_u, B)

    f_hist = jnp.zeros((5, NP_, NH), jnp.float32).at[0].set(f0)
    g_hist = jnp.zeros((5, NP_, NH), jnp.float32).at[0].set(f0)
    M = jnp.zeros((5, 5), jnp.float32).at[0, 0].set(jnp.sum(d0[0]))

    def step(alpha5, sl):
        a8 = jnp.zeros((8, NH), jnp.float32).at[:5, :].set(alpha5[:, None])
        xk, y = _prop_call(f_hist, a8, Wp)
        p = _spmm(y, pre)
        z, gn, d = _comb_call(p, aub, xk, g_hist)
        d6 = jnp.sum(d, axis=1)[:6]
        newrow = d6[:5].at[sl].set(d6[5])
        return z, gn, newrow

    alpha5 = jnp.zeros((5,), jnp.float32).at[0].set(1.0)
    z, gn, newrow = step(alpha5, 1)
    f_hist = f_hist.at[1].set(z)
    g_hist = g_hist.at[1].set(gn)
    M = M.at[1, :].set(newrow).at[:, 1].set(newrow)

    for k in range(2, THRESHOLD):
        nn = min(k, 5)
        sl = k % 5
        H = jnp.zeros((nn + 1, nn + 1), jnp.float32)
        H = H.at[0, 1:].set(1.0).at[1:, 0].set(1.0)
        H = H.at[1:, 1:].set(M[:nn, :nn] + LAM * jnp.eye(nn, dtype=jnp.float32))
        yv = jnp.zeros((nn + 1,), jnp.float32).at[0].set(1.0)
        alpha = jnp.linalg.solve(H, yv)[1:]
        alpha5 = jnp.zeros((5,), jnp.float32).at[:nn].set(alpha)
        z, gn, newrow = step(alpha5, sl)
        f_hist = f_hist.at[sl].set(z)
        g_hist = g_hist.at[sl].set(gn)
        M = M.at[sl, :].set(newrow).at[:, sl].set(newrow)

    z_star = f_hist[4]
    vwt = jnp.zeros((NH, NH), jnp.float32).at[:, :NCLASS].set(V_w.T)
    labels = _mm_call(z_star, vwt)
    return labels[:N, :NCLASS], z_star[:N]


# speedup vs baseline: 9.0986x; 1.2540x over previous
import functools

import jax
import jax.numpy as jnp
from jax import lax
from jax.experimental import pallas as pl
from jax.experimental.pallas import tpu as pltpu
from jax.experimental.pallas import tpu_sc as plsc

N = 10000
E = 160000
NH = 128
NCLASS = 16
KAPPA = 0.99
THRESHOLD = 20
LAM = 1e-4

NP_ = 10240
TILES = 32
RPT = NP_ // TILES
CHUNK = 256
EA = E + TILES * CHUNK
BLK = 512

@functools.cache
def _mesh():
    return plsc.VectorSubcoreMesh(
        core_axis_name="c", subcore_axis_name="s", num_cores=2, num_subcores=16
    )


HALF = 128


def _spmm_body(y_hbm, pack_hbm, meta_hbm, zeros_hbm,
               out_hbm, ebA, ebB, ebC, ebD, ybA, ybB, ybC, ybD, acc, mbuf,
               semGA, semGB, semGC, semGD):
    c = lax.axis_index("c")
    s = lax.axis_index("s")
    wid = s * 2 + c

    pltpu.sync_copy(meta_hbm.at[wid], mbuf)
    mv = mbuf[...]
    h_lo = mv[0] * 2
    h_hi = mv[1] * 2

    pltpu.sync_copy(zeros_hbm, acc)

    iota = lax.iota(jnp.int32, 16)

    def compute(eb, yb):
        def group_body(g, _):
            vv = plsc.bitcast(eb[1, pl.ds(g * 16, 16)], jnp.float32)
            lr = eb[2, pl.ds(g * 16, 16)]
            eids = g * 16 + iota
            for rot in range(0, 16, 2):
                lanes = [lax.bitwise_and(iota + rot + d, 15) for d in (0, 1)]
                xs = []
                for lane in lanes:
                    for j in range(8):
                        xs.append(
                            plsc.load_gather(yb, [eids, j * 16 + lane]) * vv)
                k = 0
                for lane in lanes:
                    for j in range(8):
                        plsc.addupdate_scatter(acc, [lr, j * 16 + lane], xs[k])
                        k += 1
            return 0

        lax.fori_loop(0, HALF // 16, group_body, 0)

    def fetch(h, eb, yb, sem):
        pltpu.sync_copy(pack_hbm.at[h], eb)
        return pltpu.async_copy(y_hbm.at[eb.at[0]], yb, sem)

    ebs = [ebA, ebB, ebC, ebD]
    ybs = [ybA, ybB, ybC, ybD]
    sems = [semGA, semGB, semGC, semGD]

    @pl.when(h_lo < h_hi)
    def _():
        for b in range(4):
            if b == 0:
                fetch(h_lo, ebs[0], ybs[0], sems[0])
            else:
                @pl.when(h_lo + b < h_hi)
                def _(b=b):
                    fetch(h_lo + b, ebs[b], ybs[b], sems[b])

        nquads = (h_hi - h_lo + 3) // 4

        def quad_body(i, _):
            for b in range(4):
                h = h_lo + 4 * i + b

                def phase(b=b, h=h):
                    pltpu.make_async_copy(
                        y_hbm.at[pl.ds(0, HALF)], ybs[b], sems[b]).wait()
                    compute(ebs[b], ybs[b])

                    @pl.when(h + 4 < h_hi)
                    def _():
                        fetch(h + 4, ebs[b], ybs[b], sems[b])

                if b == 0:
                    phase()
                else:
                    pl.when(h < h_hi)(phase)
            return 0

        lax.fori_loop(0, nquads, quad_body, 0)

    pltpu.sync_copy(acc, out_hbm.at[pl.ds(wid * RPT, RPT)])


@functools.cache
def _spmm_kernel():
    return pl.kernel(
        _spmm_body,
        out_type=jax.ShapeDtypeStruct((NP_, NH), jnp.float32),
        mesh=_mesh(),
        scratch_types=[
            pltpu.VMEM((3, HALF), jnp.int32),
            pltpu.VMEM((3, HALF), jnp.int32),
            pltpu.VMEM((3, HALF), jnp.int32),
            pltpu.VMEM((3, HALF), jnp.int32),
            pltpu.VMEM((HALF, NH), jnp.float32),
            pltpu.VMEM((HALF, NH), jnp.float32),
            pltpu.VMEM((HALF, NH), jnp.float32),
            pltpu.VMEM((HALF, NH), jnp.float32),
            pltpu.VMEM((RPT, NH), jnp.float32),
            pltpu.VMEM((16,), jnp.int32),
            pltpu.SemaphoreType.DMA,
            pltpu.SemaphoreType.DMA,
            pltpu.SemaphoreType.DMA,
            pltpu.SemaphoreType.DMA,
        ],
        compiler_params=pltpu.CompilerParams(needs_layout_passes=False),
    )


def _spmm_call(*args):
    return _spmm_kernel()(*args)


PROWS = NP_ // 16
PCAP = 64
_POWER_ARG_SHAPES = [
    (((EA // CHUNK + PCAP) * 4 * CHUNK,), jnp.int32),
    ((16, 16), jnp.int32),
    ((NP_,), jnp.float32),
]


def _power_body(pack_hbm, pmeta_hbm, v0_hbm, wout_hbm, vout_hbm,
                epack, sbuf, vref, wref, vseg, maxb, mb2,
                mbuf, sh_s, sh_v, sem):
    c = lax.axis_index("c")
    sid = lax.axis_index("s")

    @pl.when(c == 0)
    def _():
        pltpu.sync_copy(pmeta_hbm.at[sid], mbuf)
        mv = mbuf[...]
        ch_lo = mv[0]
        ch_hi = mv[1]
        pltpu.sync_copy(v0_hbm, vref)
        pltpu.sync_copy(pack_hbm.at[pl.ds(ch_lo * (4 * CHUNK), PCAP * 4 * CHUNK)], epack)

        iota = lax.iota(jnp.int32, 16)
        not15 = iota != 15
        is15 = iota == 15

        def do_group(ref, base, g):
            cg = ref[pl.ds(base + g * 16, 16)]
            av = plsc.bitcast(ref[pl.ds(base + CHUNK + g * 16, 16)], jnp.float32)
            lr = ref[pl.ds(base + 2 * CHUNK + g * 16, 16)]
            rn = ref[pl.ds(base + 3 * CHUNK + g * 16, 16)]
            x = plsc.load_gather(vref, [cg])
            cs = plsc.cumsum(av * x)
            chg = lr != rn
            plsc.addupdate_scatter(wref, [lr], cs, mask=chg | is15)
            plsc.addupdate_scatter(wref, [rn], -cs, mask=chg & not15)

        def matvec():
            def zero_body(i, _):
                wref[pl.ds(i * 16, 16)] = jnp.zeros((16,), jnp.float32)
                return 0
            lax.fori_loop(0, (PROWS + 16) // 16, zero_body, 0)

            def chunk_body(ci, _):
                @pl.when(ci < PCAP)
                def _():
                    def group_body(g2, _):
                        do_group(epack, ci * (4 * CHUNK), 2 * g2)
                        do_group(epack, ci * (4 * CHUNK), 2 * g2 + 1)
                        return 0
                    lax.fori_loop(0, CHUNK // 32, group_body, 0)

                @pl.when(ci >= PCAP)
                def _():
                    pltpu.sync_copy(
                        pack_hbm.at[pl.ds((ch_lo + ci) * (4 * CHUNK),
                                          4 * CHUNK)], sbuf)

                    def group_body(g2, _):
                        do_group(sbuf, 0, 2 * g2)
                        do_group(sbuf, 0, 2 * g2 + 1)
                        return 0
                    lax.fori_loop(0, CHUNK // 32, group_body, 0)
                return 0

            lax.fori_loop(0, ch_hi - ch_lo, chunk_body, 0)

        def iter_body(_, __):
            matvec()
            def max_body(i, m):
                return jnp.maximum(m, jnp.abs(wref[pl.ds(i * 16, 16)]))
            m = lax.fori_loop(0, PROWS // 16, max_body,
                              jnp.zeros((16,), jnp.float32))
            maxb[...] = jnp.full((16,), jnp.max(m), jnp.float32)
            pltpu.sync_copy(maxb, sh_s.at[sid])
            plsc.subcore_barrier()
            pltpu.sync_copy(sh_s, mb2)

            def gmax_body(t, m):
                return jnp.maximum(m, mb2[t])
            gm = lax.fori_loop(0, 16, gmax_body, jnp.zeros((16,), jnp.float32))
            sv = jnp.full((16,), jnp.max(gm), jnp.float32)
            invv = jnp.full((16,), 1.0, jnp.float32) / jnp.maximum(sv, 1e-30)

            def scale_body(i, _):
                vseg[pl.ds(i * 16, 16)] = wref[pl.ds(i * 16, 16)] * invv
                return 0
            lax.fori_loop(0, PROWS // 16, scale_body, 0)
            pltpu.sync_copy(vseg, sh_v.at[pl.ds(sid * PROWS, PROWS)])
            plsc.subcore_barrier()
            pltpu.sync_copy(sh_v, vref)
            plsc.subcore_barrier()
            return 0

        lax.fori_loop(0, 50, iter_body, 0)
        matvec()

        def out_body(i, _):
            vseg[pl.ds(i * 16, 16)] = wref[pl.ds(i * 16, 16)]
            return 0
        lax.fori_loop(0, PROWS // 16, out_body, 0)
        pltpu.sync_copy(vseg, wout_hbm.at[pl.ds(sid * PROWS, PROWS)])
        pltpu.sync_copy(vref.at[pl.ds(sid * PROWS, PROWS)],
                        vout_hbm.at[pl.ds(sid * PROWS, PROWS)])


@functools.cache
def _power_kernel():
    return pl.kernel(
        _power_body,
        out_type=(
            jax.ShapeDtypeStruct((NP_,), jnp.float32),
            jax.ShapeDtypeStruct((NP_,), jnp.float32),
        ),
        mesh=_mesh(),
        scratch_types=[
            pltpu.VMEM((PCAP * 4 * CHUNK,), jnp.int32),
            pltpu.VMEM((4 * CHUNK,), jnp.int32),
            pltpu.VMEM((NP_,), jnp.float32),
            pltpu.VMEM((PROWS + 16,), jnp.float32),
            pltpu.VMEM((PROWS,), jnp.float32),
            pltpu.VMEM((16,), jnp.float32),
            pltpu.VMEM((16, 16), jnp.float32),
            pltpu.VMEM((16,), jnp.int32),
            pltpu.VMEM_SHARED((16, 16), jnp.float32),
            pltpu.VMEM_SHARED((NP_,), jnp.float32),
            pltpu.SemaphoreType.DMA,
        ],
        compiler_params=pltpu.CompilerParams(needs_layout_passes=False),
    )


def _power_call(*args):
    return _power_kernel()(*args)


def _prop_body(f_ref, a_ref, w_ref, xk_ref, y_ref):
    F = f_ref[...]
    al = a_ref[...]
    xk = jnp.sum(F * al[:5][:, None, :], axis=0)
    xk_ref[...] = xk
    y_ref[...] = jnp.dot(xk, w_ref[...], preferred_element_type=jnp.float32)


def _prop_call(f_hist, a8, wp):
    return pl.pallas_call(
        _prop_body,
        grid=(NP_ // BLK,),
        in_specs=[
            pl.BlockSpec((5, BLK, NH), lambda i: (0, i, 0)),
            pl.BlockSpec((8, NH), lambda i: (0, 0)),
            pl.BlockSpec((NH, NH), lambda i: (0, 0)),
        ],
        out_specs=[
            pl.BlockSpec((BLK, NH), lambda i: (i, 0)),
            pl.BlockSpec((BLK, NH), lambda i: (i, 0)),
        ],
        out_shape=[
            jax.ShapeDtypeStruct((NP_, NH), jnp.float32),
            jax.ShapeDtypeStruct((NP_, NH), jnp.float32),
        ],
    )(f_hist, a8, wp)


def _comb_body(p_ref, aub_ref, xk_ref, g_ref, z_ref, gn_ref, d_ref):
    z = jnp.maximum(p_ref[...] + aub_ref[...], 0.0)
    gnew = z - xk_ref[...]
    z_ref[...] = z
    gn_ref[...] = gnew
    G = g_ref[...]
    parts = jnp.sum(G * gnew[None], axis=1)
    selfp = jnp.sum(gnew * gnew, axis=0)[None]
    d = jnp.concatenate([parts, selfp, jnp.zeros((2, NH), jnp.float32)], axis=0)

    @pl.when(pl.program_id(0) == 0)
    def _():
        d_ref[...] = jnp.zeros_like(d_ref)

    d_ref[...] += d


def _comb_call(p, aub, xk, g_hist):
    return pl.pallas_call(
        _comb_body,
        grid=(NP_ // BLK,),
        in_specs=[
            pl.BlockSpec((BLK, NH), lambda i: (i, 0)),
            pl.BlockSpec((BLK, NH), lambda i: (i, 0)),
            pl.BlockSpec((BLK, NH), lambda i: (i, 0)),
            pl.BlockSpec((5, BLK, NH), lambda i: (0, i, 0)),
        ],
        out_specs=[
            pl.BlockSpec((BLK, NH), lambda i: (i, 0)),
            pl.BlockSpec((BLK, NH), lambda i: (i, 0)),
            pl.BlockSpec((8, NH), lambda i: (0, 0)),
        ],
        out_shape=[
            jax.ShapeDtypeStruct((NP_, NH), jnp.float32),
            jax.ShapeDtypeStruct((NP_, NH), jnp.float32),
            jax.ShapeDtypeStruct((8, NH), jnp.float32),
        ],
    )(p, aub, xk, g_hist)


def _init_body(pu_ref, b_ref, aub_ref, f0_ref, d_ref):
    aub = jnp.dot(pu_ref[...], b_ref[...], preferred_element_type=jnp.float32)
    f0 = jnp.maximum(aub, 0.0)
    aub_ref[...] = aub
    f0_ref[...] = f0
    d = jnp.concatenate(
        [jnp.sum(f0 * f0, axis=0)[None], jnp.zeros((7, NH), jnp.float32)], axis=0
    )

    @pl.when(pl.program_id(0) == 0)
    def _():
        d_ref[...] = jnp.zeros_like(d_ref)

    d_ref[...] += d


def _init_call(pu, b):
    return pl.pallas_call(
        _init_body,
        grid=(NP_ // BLK,),
        in_specs=[
            pl.BlockSpec((BLK, NH), lambda i: (i, 0)),
            pl.BlockSpec((NH, NH), lambda i: (0, 0)),
        ],
        out_specs=[
            pl.BlockSpec((BLK, NH), lambda i: (i, 0)),
            pl.BlockSpec((BLK, NH), lambda i: (i, 0)),
            pl.BlockSpec((8, NH), lambda i: (0, 0)),
        ],
        out_shape=[
            jax.ShapeDtypeStruct((NP_, NH), jnp.float32),
            jax.ShapeDtypeStruct((NP_, NH), jnp.float32),
            jax.ShapeDtypeStruct((8, NH), jnp.float32),
        ],
    )(pu, b)


def _proj_body(w_ref, kap_ref, out_ref):
    W = w_ref[...]
    kapc = kap_ref[...][0:1, 0:1]
    a = jnp.abs(W)
    s = jnp.sum(a, axis=1, keepdims=True)
    hi0 = jnp.max(a, axis=1, keepdims=True)

    def bis(_, lh):
        lo, hi = lh
        mid = 0.5 * (lo + hi)
        t = jnp.sum(jnp.maximum(a - mid, 0.0), axis=1, keepdims=True)
        pred = t > kapc
        return jnp.where(pred, mid, lo), jnp.where(pred, hi, mid)

    lo, hi = lax.fori_loop(0, 60, bis, (jnp.zeros_like(s), hi0))
    theta = 0.5 * (lo + hi)
    proj = jnp.sign(W) * jnp.maximum(a - theta, 0.0)
    out_ref[...] = jnp.where(s > kapc, proj, W)


def _proj_call(w, kap):
    return pl.pallas_call(
        _proj_body,
        in_specs=[
            pl.BlockSpec((NH, NH), lambda: (0, 0)),
            pl.BlockSpec((8, NH), lambda: (0, 0)),
        ],
        out_specs=pl.BlockSpec((NH, NH), lambda: (0, 0)),
        out_shape=jax.ShapeDtypeStruct((NH, NH), jnp.float32),
    )(w, kap)


def _mm_body(x_ref, w_ref, o_ref):
    o_ref[...] = jnp.dot(x_ref[...], w_ref[...], preferred_element_type=jnp.float32)


def _mm_call(x, w):
    return pl.pallas_call(
        _mm_body,
        grid=(NP_ // BLK,),
        in_specs=[
            pl.BlockSpec((BLK, NH), lambda i: (i, 0)),
            pl.BlockSpec((NH, NH), lambda i: (0, 0)),
        ],
        out_specs=pl.BlockSpec((BLK, NH), lambda i: (i, 0)),
        out_shape=jax.ShapeDtypeStruct((NP_, NH), jnp.float32),
    )(x, w)


def _preprocess(edge_index, a_values):
    row = edge_index[0].astype(jnp.int32)
    col = edge_index[1].astype(jnp.int32)
    order = jnp.argsort(row)
    rs = row[order]
    cls = col[order]
    vs = a_values[order]

    t_e = rs // RPT
    cnt = jnp.bincount(t_e, length=TILES)
    nch = (cnt + CHUNK - 1) // CHUNK
    choff = jnp.concatenate([jnp.zeros((1,), jnp.int32),
                             jnp.cumsum(nch).astype(jnp.int32)])
    seg_start = choff * CHUNK
    cum_cnt = jnp.concatenate([jnp.zeros((1,), jnp.int32),
                               jnp.cumsum(cnt).astype(jnp.int32)])
    pos = seg_start[t_e] + (jnp.arange(E, dtype=jnp.int32) - cum_cnt[t_e])

    cols_p = jnp.zeros((EA,), jnp.int32).at[pos].set(cls)
    vals_p = jnp.zeros((EA,), jnp.float32).at[pos].set(vs)
    lrow_p = jnp.zeros((EA,), jnp.int32).at[pos].set(rs - t_e * RPT)

    meta = jnp.zeros((TILES, 16), jnp.int32)
    meta = meta.at[:, 0].set(choff[:-1])
    meta = meta.at[:, 1].set(choff[1:])

    vbits = lax.bitcast_convert_type(vals_p, jnp.int32)
    pack_spmm = jnp.stack(
        [cols_p.reshape(EA // HALF, HALF),
         vbits.reshape(EA // HALF, HALF),
         lrow_p.reshape(EA // HALF, HALF)], axis=1)

    slot = jnp.arange(EA, dtype=jnp.int32)
    tile_of_slot = jnp.searchsorted(seg_start[1:], slot, side="right").astype(jnp.int32)
    lr640 = (640 + (tile_of_slot % 2)).at[pos].set(rs - (t_e // 2) * PROWS)
    rn640 = jnp.roll(lr640, -1)
    abits = lax.bitcast_convert_type(jnp.abs(vals_p), jnp.int32)
    pack_pow = jnp.concatenate(
        [cols_p.reshape(EA // CHUNK, CHUNK),
         abits.reshape(EA // CHUNK, CHUNK),
         lr640.reshape(EA // CHUNK, CHUNK),
         rn640.reshape(EA // CHUNK, CHUNK)], axis=1)
    pack_pow = jnp.concatenate(
        [pack_pow, jnp.zeros((PCAP, 4 * CHUNK), jnp.int32)], axis=0).reshape(-1)
    pmeta = jnp.zeros((16, 16), jnp.int32)
    pmeta = pmeta.at[:, 0].set(choff[0:32:2]).at[:, 1].set(choff[2::2])
    v0 = jnp.concatenate([jnp.full((N,), 0.01, jnp.float32),
                          jnp.zeros((NP_ - N,), jnp.float32)])
    power_args = (pack_pow, pmeta, v0)
    return (pack_spmm, meta), power_args


def _spmm(y, pre):
    pack_spmm, meta = pre
    zeros = jnp.zeros((RPT, NH), jnp.float32)
    return _spmm_call(y, pack_spmm, meta, zeros)


def kernel(U, edge_index, A_values, W, B, V_w):
    pre, power_args = _preprocess(edge_index, A_values)

    w_out, v_out = _power_call(*power_args)
    rho = (jnp.linalg.norm(w_out)
           / jnp.maximum(jnp.linalg.norm(v_out), 1e-30)) + 1e-5
    kap_eff = KAPPA / rho
    kap_arr = jnp.full((8, NH), kap_eff, jnp.float32)
    Wp = _proj_call(W, kap_arr)

    u_pad = jnp.concatenate([U, jnp.zeros((NP_ - N, NH), jnp.float32)], axis=0)
    p_u = _spmm(u_pad, pre)
    aub, f0, d0 = _init_call(p_u, B)

    f_hist = jnp.zeros((5, NP_, NH), jnp.float32).at[0].set(f0)
    g_hist = jnp.zeros((5, NP_, NH), jnp.float32).at[0].set(f0)
    M = jnp.zeros((5, 5), jnp.float32).at[0, 0].set(jnp.sum(d0[0]))

    def step(alpha5, sl):
        a8 = jnp.zeros((8, NH), jnp.float32).at[:5, :].set(alpha5[:, None])
        xk, y = _prop_call(f_hist, a8, Wp)
        p = _spmm(y, pre)
        z, gn, d = _comb_call(p, aub, xk, g_hist)
        d6 = jnp.sum(d, axis=1)[:6]
        newrow = d6[:5].at[sl].set(d6[5])
        return z, gn, newrow

    alpha5 = jnp.zeros((5,), jnp.float32).at[0].set(1.0)
    z, gn, newrow = step(alpha5, 1)
    f_hist = f_hist.at[1].set(z)
    g_hist = g_hist.at[1].set(gn)
    M = M.at[1, :].set(newrow).at[:, 1].set(newrow)

    for k in range(2, THRESHOLD):
        nn = min(k, 5)
        sl = k % 5
        H = jnp.zeros((nn + 1, nn + 1), jnp.float32)
        H = H.at[0, 1:].set(1.0).at[1:, 0].set(1.0)
        H = H.at[1:, 1:].set(M[:nn, :nn] + LAM * jnp.eye(nn, dtype=jnp.float32))
        yv = jnp.zeros((nn + 1,), jnp.float32).at[0].set(1.0)
        alpha = jnp.linalg.solve(H, yv)[1:]
        alpha5 = jnp.zeros((5,), jnp.float32).at[:nn].set(alpha)
        z, gn, newrow = step(alpha5, sl)
        f_hist = f_hist.at[sl].set(z)
        g_hist = g_hist.at[sl].set(gn)
        M = M.at[sl, :].set(newrow).at[:, sl].set(newrow)

    z_star = f_hist[4]
    vwt = jnp.zeros((NH, NH), jnp.float32).at[:, :NCLASS].set(V_w.T)
    labels = _mm_call(z_star, vwt)
    return labels[:N, :NCLASS], z_star[:N]
